# Initial kernel scaffold; baseline (speedup 1.0000x reference)
import jax
import jax.numpy as jnp
from jax.experimental import pallas as pl


def kernel(x_cell, x_global, pos_cell, edge_index, batch_cell, batch_global, Ws0, Wn0, Wgl0, bc0, Gi0, Gs0, bg0, Ws1, Wn1, Wgl1, bc1, Gi1, Gs1, bg1, l1W, l1b, l2W, l2b, l3W, l3b):
    raise NotImplementedError("write your pallas kernel here")


# speedup vs baseline: 4.8309x; 4.8309x over previous
import functools

import jax
import jax.numpy as jnp
from jax import lax
from jax.experimental import pallas as pl
from jax.experimental.pallas import tpu as pltpu
from jax.experimental.pallas import tpu_sc as plsc

N = 10000
E = 320000
B = 32
D = 128
H = 128
OUT = 2

NC = 2
NS = 16
EPC = E // NC
EPT = EPC // NS
CH = 128
NFULL = EPT // CH
TAIL = EPT - NFULL * CH
RPT = 624
RTAIL = N - RPT * NS
ZB = 208
DW = 16

_f32 = jnp.float32


HD = D // 2


def _msg_body(with_deg, *refs):
  if with_deg:
    (table, src, dst, out_msg, out_deg,
     idx_s, idx_d, idx_g, idx_s2, idx_d2, idx_g2, rows, ones, zbuf, zdeg,
     acc, dacc, sem) = refs
  else:
    (table, src, dst, out_msg,
     idx_s, idx_d, idx_g, idx_s2, idx_d2, idx_g2, rows, ones, zbuf, zdeg,
     acc, sem) = refs
    out_deg = None
    dacc = None

  cid = lax.axis_index("c")
  sid = lax.axis_index("s")

  z16 = jnp.zeros((16,), _f32)
  o16 = jnp.ones((16,), _f32)

  @pl.loop(0, ZB)
  def _(r):
    @pl.loop(0, HD // 16)
    def _(c):
      zbuf[r, pl.ds(c * 16, 16)] = z16

  @pl.loop(0, ZB)
  def _(r):
    zdeg[r, :] = z16

  @pl.loop(0, CH)
  def _(r):
    ones[r, :] = o16

  rbase = sid * RPT
  ebase = cid * EPC + sid * EPT

  for hf in range(2):
    deg_now = with_deg and hf == 0

    @pl.loop(0, 3)
    def _(j):
      pltpu.sync_copy(zbuf, acc.at[pl.ds(rbase + j * ZB, ZB)])
      if deg_now:
        pltpu.sync_copy(zdeg, dacc.at[pl.ds(rbase + j * ZB, ZB)])

    @pl.when(sid == 0)
    def _():
      pltpu.sync_copy(zbuf.at[pl.ds(0, RTAIL)], acc.at[pl.ds(RPT * NS, RTAIL)])
      if deg_now:
        pltpu.sync_copy(zdeg.at[pl.ds(0, RTAIL)],
                        dacc.at[pl.ds(RPT * NS, RTAIL)])

    plsc.subcore_barrier()

    @pl.loop(0, NFULL)
    def _(i):
      off = pl.multiple_of(ebase + i * CH, 8)
      pltpu.sync_copy(src.at[pl.ds(off, CH)], idx_s)
      pltpu.sync_copy(dst.at[pl.ds(off, CH)], idx_d)

      @pl.loop(0, CH // 16)
      def _(j):
        idx_g[pl.ds(j * 16, 16)] = idx_s[pl.ds(j * 16, 16)] * 2 + hf

      pltpu.async_copy(table.at[idx_g], rows, sem).wait()
      pltpu.sync_copy(rows, acc.at[idx_d], add=True)
      if deg_now:
        pltpu.sync_copy(ones, dacc.at[idx_d], add=True)

    toff = pl.multiple_of(ebase + NFULL * CH, 8)
    pltpu.sync_copy(src.at[pl.ds(toff, TAIL)], idx_s2)
    pltpu.sync_copy(dst.at[pl.ds(toff, TAIL)], idx_d2)
    idx_g2[...] = idx_s2[...] * 2 + hf
    pltpu.async_copy(table.at[idx_g2], rows.at[pl.ds(0, TAIL)], sem).wait()
    pltpu.sync_copy(rows.at[pl.ds(0, TAIL)], acc.at[idx_d2], add=True)
    if deg_now:
      pltpu.sync_copy(ones.at[pl.ds(0, TAIL)], dacc.at[idx_d2], add=True)

    plsc.subcore_barrier()

    pltpu.sync_copy(acc.at[pl.ds(rbase, RPT)],
                    out_msg.at[cid, hf, pl.ds(rbase, RPT)])
    if deg_now:
      pltpu.sync_copy(dacc.at[pl.ds(rbase, RPT)],
                      out_deg.at[cid, pl.ds(rbase, RPT)])

    @pl.when(sid == 0)
    def _():
      pltpu.sync_copy(acc.at[pl.ds(RPT * NS, RTAIL)],
                      out_msg.at[cid, hf, pl.ds(RPT * NS, RTAIL)])
      if deg_now:
        pltpu.sync_copy(dacc.at[pl.ds(RPT * NS, RTAIL)],
                        out_deg.at[cid, pl.ds(RPT * NS, RTAIL)])


def _make_msg_kernel(with_deg):
  mesh = plsc.VectorSubcoreMesh(core_axis_name="c", subcore_axis_name="s",
                                num_cores=NC, num_subcores=NS)
  out_type = [jax.ShapeDtypeStruct((NC, 2, N, HD), _f32)]
  if with_deg:
    out_type.append(jax.ShapeDtypeStruct((NC, N, DW), _f32))
  scratch = [
      pltpu.VMEM((CH,), jnp.int32),
      pltpu.VMEM((CH,), jnp.int32),
      pltpu.VMEM((CH,), jnp.int32),
      pltpu.VMEM((TAIL,), jnp.int32),
      pltpu.VMEM((TAIL,), jnp.int32),
      pltpu.VMEM((TAIL,), jnp.int32),
      pltpu.VMEM((CH, HD), _f32),
      pltpu.VMEM((CH, DW), _f32),
      pltpu.VMEM((ZB, HD), _f32),
      pltpu.VMEM((ZB, DW), _f32),
      pltpu.VMEM_SHARED((N, HD), _f32),
  ]
  if with_deg:
    scratch.append(pltpu.VMEM_SHARED((N, DW), _f32))
  scratch.append(pltpu.SemaphoreType.DMA)
  return pl.kernel(functools.partial(_msg_body, with_deg),
                   out_type=tuple(out_type), mesh=mesh,
                   scratch_types=scratch,
                   compiler_params=pltpu.CompilerParams(
                       use_tc_tiling_on_sc=False))


_msg_deg_kernel = _make_msg_kernel(True)
_msg_kernel = _make_msg_kernel(False)

BN = 1000
NG = N // BN


def _dense_body(h, msgp, degp, bcell, g, ws, wn, wgl, bc, gi, gs, bg,
                h_new, g_new, pooled_acc, cnt_acc):
  i = pl.program_id(0)
  hv = h[...]
  msg = jnp.concatenate([msgp[0, 0] + msgp[1, 0],
                         msgp[0, 1] + msgp[1, 1]], axis=1)
  deg = jnp.maximum(degp[0, :, 0:1] + degp[1, :, 0:1], 1.0)
  msg = msg / deg
  gv = g[...]
  gw = jnp.dot(gv, wgl[...], preferred_element_type=_f32)
  s = (bcell[...] == lax.broadcasted_iota(jnp.int32, (BN, B), 1)
       ).astype(_f32)
  hn = (jnp.dot(hv, ws[...], preferred_element_type=_f32)
        + jnp.dot(msg, wn[...], preferred_element_type=_f32)
        + jnp.dot(s, gw, preferred_element_type=_f32)
        + bc[...])
  hn = jnp.maximum(hn, 0.0)
  h_new[...] = hn

  @pl.when(i == 0)
  def _():
    pooled_acc[...] = jnp.zeros((B, H), _f32)
    cnt_acc[...] = jnp.zeros((B, H), _f32)

  dn = (((0,), (0,)), ((), ()))
  pooled_acc[...] += lax.dot_general(s, hn, dn,
                                     preferred_element_type=_f32)
  cnt_acc[...] += lax.dot_general(s, jnp.ones((BN, H), _f32), dn,
                                  preferred_element_type=_f32)

  @pl.when(i == NG - 1)
  def _():
    pooled = pooled_acc[...] / jnp.maximum(cnt_acc[...], 1.0)
    gn = (jnp.dot(pooled, gi[...], preferred_element_type=_f32)
          + jnp.dot(gv, gs[...], preferred_element_type=_f32)
          + bg[...])
    g_new[...] = jnp.maximum(gn, 0.0)


_dense_call = pl.pallas_call(
    _dense_body,
    grid=(NG,),
    in_specs=[
        pl.BlockSpec((BN, D), lambda i: (i, 0)),
        pl.BlockSpec((NC, 2, BN, HD), lambda i: (0, 0, i, 0)),
        pl.BlockSpec((NC, BN, DW), lambda i: (0, i, 0)),
        pl.BlockSpec((BN, 1), lambda i: (i, 0)),
        pl.BlockSpec((B, H), lambda i: (0, 0)),
        pl.BlockSpec((H, H), lambda i: (0, 0)),
        pl.BlockSpec((H, H), lambda i: (0, 0)),
        pl.BlockSpec((H, H), lambda i: (0, 0)),
        pl.BlockSpec((1, H), lambda i: (0, 0)),
        pl.BlockSpec((H, H), lambda i: (0, 0)),
        pl.BlockSpec((H, H), lambda i: (0, 0)),
        pl.BlockSpec((1, H), lambda i: (0, 0)),
    ],
    out_specs=[
        pl.BlockSpec((BN, H), lambda i: (i, 0)),
        pl.BlockSpec((B, H), lambda i: (0, 0)),
    ],
    out_shape=[
        jax.ShapeDtypeStruct((N, H), _f32),
        jax.ShapeDtypeStruct((B, H), _f32),
    ],
    scratch_shapes=[
        pltpu.VMEM((B, H), _f32),
        pltpu.VMEM((B, H), _f32),
    ],
)


def _head_body(h, pos, bcell, g, l1w, l1b, l2w, l2b, l3w, l3b,
               out, r_acc):
  i = pl.program_id(0)
  p = pos[...]
  x0 = p[:, 0:1] - 600.0
  x1 = p[:, 1:2] - 600.0
  dd = x1 - x0
  nd = x1 + x0
  lab = ((dd > 0.0).astype(jnp.int32)
         + 2 * (nd < 0.0).astype(jnp.int32))
  c = lab * B + bcell[...]
  cm = (c == lax.broadcasted_iota(jnp.int32, (BN, 4 * B), 1)).astype(_f32)

  @pl.when(i == 0)
  def _():
    r_acc[...] = jnp.zeros((4 * B, H), _f32)

  dn = (((0,), (0,)), ((), ()))
  r_acc[...] += lax.dot_general(cm, h[...], dn, preferred_element_type=_f32)

  @pl.when(i == NG - 1)
  def _():
    r = r_acc[...]
    gv = g[...]
    acc = jnp.dot(gv, l1w[pl.ds(4 * H, H), :], preferred_element_type=_f32)
    for q in range(4):
      acc += jnp.dot(r[q * B:(q + 1) * B, :], l1w[pl.ds(q * H, H), :],
                     preferred_element_type=_f32)
    y1 = jnp.maximum(acc + l1b[...], 0.0)
    y2 = jnp.maximum(jnp.dot(y1, l2w[...], preferred_element_type=_f32)
                     + l2b[...], 0.0)
    out[...] = (jnp.dot(y2, l3w[...], preferred_element_type=_f32)
                + l3b[...])


_head_call = pl.pallas_call(
    _head_body,
    grid=(NG,),
    in_specs=[
        pl.BlockSpec((BN, H), lambda i: (i, 0)),
        pl.BlockSpec((BN, 2), lambda i: (i, 0)),
        pl.BlockSpec((BN, 1), lambda i: (i, 0)),
        pl.BlockSpec((B, H), lambda i: (0, 0)),
        pl.BlockSpec((5 * H, 2 * H), lambda i: (0, 0)),
        pl.BlockSpec((1, 2 * H), lambda i: (0, 0)),
        pl.BlockSpec((2 * H, H), lambda i: (0, 0)),
        pl.BlockSpec((1, H), lambda i: (0, 0)),
        pl.BlockSpec((H, OUT), lambda i: (0, 0)),
        pl.BlockSpec((1, OUT), lambda i: (0, 0)),
    ],
    out_specs=pl.BlockSpec((B, OUT), lambda i: (0, 0)),
    out_shape=jax.ShapeDtypeStruct((B, OUT), _f32),
    scratch_shapes=[pltpu.VMEM((4 * B, H), _f32)],
)


@jax.jit
def kernel(x_cell, x_global, pos_cell, edge_index, batch_cell, batch_global,
           Ws0, Wn0, Wgl0, bc0, Gi0, Gs0, bg0,
           Ws1, Wn1, Wgl1, bc1, Gi1, Gs1, bg1,
           l1W, l1b, l2W, l2b, l3W, l3b):
  src = edge_index[0]
  dst = edge_index[1]
  bcell = batch_cell.reshape(N, 1)

  msg0, degp = _msg_deg_kernel(x_cell.reshape(2 * N, HD), src, dst)
  h1, g1 = _dense_call(x_cell, msg0, degp, bcell, x_global,
                       Ws0, Wn0, Wgl0, bc0.reshape(1, H),
                       Gi0, Gs0, bg0.reshape(1, H))
  msg1 = _msg_kernel(h1.reshape(2 * N, HD), src, dst)[0]
  h2, g2 = _dense_call(h1, msg1, degp, bcell, g1,
                       Ws1, Wn1, Wgl1, bc1.reshape(1, H),
                       Gi1, Gs1, bg1.reshape(1, H))
  return _head_call(h2, pos_cell, bcell, g2,
                    l1W, l1b.reshape(1, 2 * H), l2W, l2b.reshape(1, H),
                    l3W, l3b.reshape(1, OUT))


# speedup vs baseline: 10.1018x; 2.0911x over previous
import functools

import jax
import jax.numpy as jnp
from jax import lax
from jax.experimental import pallas as pl
from jax.experimental.pallas import tpu as pltpu
from jax.experimental.pallas import tpu_sc as plsc

N = 10000
E = 320000
B = 32
D = 128
H = 128
OUT = 2

NC = 2
NS = 16
EPC = E // NC
EPT = EPC // NS
CH = 128
NFULL = EPT // CH
TAIL = EPT - NFULL * CH
RPT = 624
RTAIL = N - RPT * NS
ZB = 208
DW = 16

_f32 = jnp.float32


HD = D // 2
NBUF = 6
OUTER = NFULL // NBUF


def _msg_body(with_deg, *refs):
  it = iter(refs)
  table = next(it)
  src = next(it)
  dst = next(it)
  out_msg = next(it)
  out_deg = next(it) if with_deg else None
  idx_s = [next(it) for _ in range(NBUF)]
  idx_d = [next(it) for _ in range(NBUF)]
  idx_g = [next(it) for _ in range(NBUF)]
  rows = [next(it) for _ in range(NBUF)]
  idx_s2 = next(it)
  idx_d2 = next(it)
  idx_g2 = next(it)
  ones = next(it)
  zbuf = next(it)
  zdeg = next(it)
  acc = next(it)
  dacc = next(it) if with_deg else None
  sem_i = [next(it) for _ in range(NBUF)]
  sem_g = [next(it) for _ in range(NBUF)]
  sem_c = [next(it) for _ in range(NBUF)]
  sem_t = next(it)

  cid = lax.axis_index("c")
  sid = lax.axis_index("s")

  z16 = jnp.zeros((16,), _f32)
  o16 = jnp.ones((16,), _f32)

  @pl.loop(0, ZB)
  def _(r):
    @pl.loop(0, HD // 16)
    def _(c):
      zbuf[r, pl.ds(c * 16, 16)] = z16

  @pl.loop(0, ZB)
  def _(r):
    zdeg[r, :] = z16

  @pl.loop(0, CH)
  def _(r):
    ones[r, :] = o16

  rbase = sid * RPT
  ebase = cid * EPC + sid * EPT

  def start_idx(b, c):
    off = pl.multiple_of(ebase + c * CH, 8)
    pltpu.async_copy(src.at[pl.ds(off, CH)], idx_s[b], sem_i[b])
    pltpu.async_copy(dst.at[pl.ds(off, CH)], idx_d[b], sem_i[b])

  def wait_idx(b):
    pltpu.make_async_copy(src.at[pl.ds(0, CH)], idx_s[b], sem_i[b]).wait()
    pltpu.make_async_copy(dst.at[pl.ds(0, CH)], idx_d[b], sem_i[b]).wait()

  for hf in range(2):
    deg_now = with_deg and hf == 0

    for b in range(NBUF):
      start_idx(b, b)

    @pl.loop(0, 3)
    def _(j):
      pltpu.sync_copy(zbuf, acc.at[pl.ds(rbase + j * ZB, ZB)])
      if deg_now:
        pltpu.sync_copy(zdeg, dacc.at[pl.ds(rbase + j * ZB, ZB)])

    @pl.when(sid == 0)
    def _():
      pltpu.sync_copy(zbuf.at[pl.ds(0, RTAIL)], acc.at[pl.ds(RPT * NS, RTAIL)])
      if deg_now:
        pltpu.sync_copy(zdeg.at[pl.ds(0, RTAIL)],
                        dacc.at[pl.ds(RPT * NS, RTAIL)])

    plsc.subcore_barrier()

    @pl.loop(0, OUTER)
    def _(i):
      base_c = i * NBUF
      for b in range(NBUF):
        wait_idx(b)

        @pl.loop(0, CH // 16)
        def _(j):
          idx_g[b][pl.ds(j * 16, 16)] = idx_s[b][pl.ds(j * 16, 16)] * 2 + hf

        pltpu.async_copy(table.at[idx_g[b]], rows[b], sem_g[b])
      for b in range(NBUF):
        pltpu.make_async_copy(table.at[pl.ds(0, CH)], rows[b],
                              sem_g[b]).wait()
        pltpu.async_copy(rows[b], acc.at[idx_d[b]], sem_c[b], add=True)
        if deg_now:
          pltpu.async_copy(ones, dacc.at[idx_d[b]], sem_c[b], add=True)
      for b in range(NBUF):
        pltpu.make_async_copy(table.at[pl.ds(0, CH)], rows[b],
                              sem_c[b]).wait()
        if deg_now:
          pltpu.make_async_copy(out_deg.at[0, pl.ds(0, CH)], ones,
                                sem_c[b]).wait()
        nc = base_c + NBUF + b

        @pl.when(nc < NFULL)
        def _():
          start_idx(b, nc)

    toff = pl.multiple_of(ebase + NFULL * CH, 8)
    pltpu.sync_copy(src.at[pl.ds(toff, TAIL)], idx_s2)
    pltpu.sync_copy(dst.at[pl.ds(toff, TAIL)], idx_d2)
    idx_g2[...] = idx_s2[...] * 2 + hf
    pltpu.async_copy(table.at[idx_g2], rows[0].at[pl.ds(0, TAIL)],
                     sem_t).wait()
    pltpu.sync_copy(rows[0].at[pl.ds(0, TAIL)], acc.at[idx_d2], add=True)
    if deg_now:
      pltpu.sync_copy(ones.at[pl.ds(0, TAIL)], dacc.at[idx_d2], add=True)

    plsc.subcore_barrier()

    pltpu.sync_copy(acc.at[pl.ds(rbase, RPT)],
                    out_msg.at[cid, hf, pl.ds(rbase, RPT)])
    if deg_now:
      pltpu.sync_copy(dacc.at[pl.ds(rbase, RPT)],
                      out_deg.at[cid, pl.ds(rbase, RPT)])

    @pl.when(sid == 0)
    def _():
      pltpu.sync_copy(acc.at[pl.ds(RPT * NS, RTAIL)],
                      out_msg.at[cid, hf, pl.ds(RPT * NS, RTAIL)])
      if deg_now:
        pltpu.sync_copy(dacc.at[pl.ds(RPT * NS, RTAIL)],
                        out_deg.at[cid, pl.ds(RPT * NS, RTAIL)])


def _make_msg_kernel(with_deg):
  mesh = plsc.VectorSubcoreMesh(core_axis_name="c", subcore_axis_name="s",
                                num_cores=NC, num_subcores=NS)
  out_type = [jax.ShapeDtypeStruct((NC, 2, N, HD), _f32)]
  if with_deg:
    out_type.append(jax.ShapeDtypeStruct((NC, N, DW), _f32))
  scratch = []
  scratch += [pltpu.VMEM((CH,), jnp.int32) for _ in range(NBUF)]
  scratch += [pltpu.VMEM((CH,), jnp.int32) for _ in range(NBUF)]
  scratch += [pltpu.VMEM((CH,), jnp.int32) for _ in range(NBUF)]
  scratch += [pltpu.VMEM((CH, HD), _f32) for _ in range(NBUF)]
  scratch += [
      pltpu.VMEM((TAIL,), jnp.int32),
      pltpu.VMEM((TAIL,), jnp.int32),
      pltpu.VMEM((TAIL,), jnp.int32),
      pltpu.VMEM((CH, DW), _f32),
      pltpu.VMEM((ZB, HD), _f32),
      pltpu.VMEM((ZB, DW), _f32),
      pltpu.VMEM_SHARED((N, HD), _f32),
  ]
  if with_deg:
    scratch.append(pltpu.VMEM_SHARED((N, DW), _f32))
  scratch += [pltpu.SemaphoreType.DMA for _ in range(3 * NBUF + 1)]
  return pl.kernel(functools.partial(_msg_body, with_deg),
                   out_type=tuple(out_type), mesh=mesh,
                   scratch_types=scratch,
                   compiler_params=pltpu.CompilerParams(
                       use_tc_tiling_on_sc=False))


_msg_deg_kernel = _make_msg_kernel(True)
_msg_kernel = _make_msg_kernel(False)

BN = 1000
NG = N // BN


def _dense_body(h, msgp, degp, bcell, g, ws, wn, wgl, bc, gi, gs, bg,
                h_new, g_new, pooled_acc, cnt_acc):
  i = pl.program_id(0)
  hv = h[...]
  msg = jnp.concatenate([msgp[0, 0] + msgp[1, 0],
                         msgp[0, 1] + msgp[1, 1]], axis=1)
  deg = jnp.maximum(degp[0, :, 0:1] + degp[1, :, 0:1], 1.0)
  msg = msg / deg
  gv = g[...]
  gw = jnp.dot(gv, wgl[...], preferred_element_type=_f32)
  s = (bcell[...] == lax.broadcasted_iota(jnp.int32, (BN, B), 1)
       ).astype(_f32)
  hn = (jnp.dot(hv, ws[...], preferred_element_type=_f32)
        + jnp.dot(msg, wn[...], preferred_element_type=_f32)
        + jnp.dot(s, gw, preferred_element_type=_f32,
                  precision=lax.Precision.HIGHEST)
        + bc[...])
  hn = jnp.maximum(hn, 0.0)
  h_new[...] = hn

  @pl.when(i == 0)
  def _():
    pooled_acc[...] = jnp.zeros((B, H), _f32)
    cnt_acc[...] = jnp.zeros((B, H), _f32)

  dn = (((0,), (0,)), ((), ()))
  pooled_acc[...] += lax.dot_general(s, hn, dn, preferred_element_type=_f32,
                                     precision=lax.Precision.HIGHEST)
  cnt_acc[...] += lax.dot_general(s, jnp.ones((BN, H), _f32), dn,
                                  preferred_element_type=_f32,
                                  precision=lax.Precision.HIGHEST)

  @pl.when(i == NG - 1)
  def _():
    pooled = pooled_acc[...] / jnp.maximum(cnt_acc[...], 1.0)
    gn = (jnp.dot(pooled, gi[...], preferred_element_type=_f32)
          + jnp.dot(gv, gs[...], preferred_element_type=_f32)
          + bg[...])
    g_new[...] = jnp.maximum(gn, 0.0)


_dense_call = pl.pallas_call(
    _dense_body,
    grid=(NG,),
    in_specs=[
        pl.BlockSpec((BN, D), lambda i: (i, 0)),
        pl.BlockSpec((NC, 2, BN, HD), lambda i: (0, 0, i, 0)),
        pl.BlockSpec((NC, BN, DW), lambda i: (0, i, 0)),
        pl.BlockSpec((BN, 1), lambda i: (i, 0)),
        pl.BlockSpec((B, H), lambda i: (0, 0)),
        pl.BlockSpec((H, H), lambda i: (0, 0)),
        pl.BlockSpec((H, H), lambda i: (0, 0)),
        pl.BlockSpec((H, H), lambda i: (0, 0)),
        pl.BlockSpec((1, H), lambda i: (0, 0)),
        pl.BlockSpec((H, H), lambda i: (0, 0)),
        pl.BlockSpec((H, H), lambda i: (0, 0)),
        pl.BlockSpec((1, H), lambda i: (0, 0)),
    ],
    out_specs=[
        pl.BlockSpec((BN, H), lambda i: (i, 0)),
        pl.BlockSpec((B, H), lambda i: (0, 0)),
    ],
    out_shape=[
        jax.ShapeDtypeStruct((N, H), _f32),
        jax.ShapeDtypeStruct((B, H), _f32),
    ],
    scratch_shapes=[
        pltpu.VMEM((B, H), _f32),
        pltpu.VMEM((B, H), _f32),
    ],
)


def _head_body(h, pos, bcell, g, l1w, l1b, l2w, l2b, l3w, l3b,
               out, r_acc):
  i = pl.program_id(0)
  p = pos[...]
  x0 = p[:, 0:1] - 600.0
  x1 = p[:, 1:2] - 600.0
  dd = x1 - x0
  nd = x1 + x0
  lab = ((dd > 0.0).astype(jnp.int32)
         + 2 * (nd < 0.0).astype(jnp.int32))
  c = lab * B + bcell[...]
  cm = (c == lax.broadcasted_iota(jnp.int32, (BN, 4 * B), 1)).astype(_f32)

  @pl.when(i == 0)
  def _():
    r_acc[...] = jnp.zeros((4 * B, H), _f32)

  dn = (((0,), (0,)), ((), ()))
  r_acc[...] += lax.dot_general(cm, h[...], dn, preferred_element_type=_f32,
                                precision=lax.Precision.HIGHEST)

  @pl.when(i == NG - 1)
  def _():
    r = r_acc[...]
    gv = g[...]
    acc = jnp.dot(gv, l1w[pl.ds(4 * H, H), :], preferred_element_type=_f32)
    for q in range(4):
      acc += jnp.dot(r[q * B:(q + 1) * B, :], l1w[pl.ds(q * H, H), :],
                     preferred_element_type=_f32)
    y1 = jnp.maximum(acc + l1b[...], 0.0)
    y2 = jnp.maximum(jnp.dot(y1, l2w[...], preferred_element_type=_f32)
                     + l2b[...], 0.0)
    out[...] = (jnp.dot(y2, l3w[...], preferred_element_type=_f32)
                + l3b[...])


_head_call = pl.pallas_call(
    _head_body,
    grid=(NG,),
    in_specs=[
        pl.BlockSpec((BN, H), lambda i: (i, 0)),
        pl.BlockSpec((BN, 2), lambda i: (i, 0)),
        pl.BlockSpec((BN, 1), lambda i: (i, 0)),
        pl.BlockSpec((B, H), lambda i: (0, 0)),
        pl.BlockSpec((5 * H, 2 * H), lambda i: (0, 0)),
        pl.BlockSpec((1, 2 * H), lambda i: (0, 0)),
        pl.BlockSpec((2 * H, H), lambda i: (0, 0)),
        pl.BlockSpec((1, H), lambda i: (0, 0)),
        pl.BlockSpec((H, OUT), lambda i: (0, 0)),
        pl.BlockSpec((1, OUT), lambda i: (0, 0)),
    ],
    out_specs=pl.BlockSpec((B, OUT), lambda i: (0, 0)),
    out_shape=jax.ShapeDtypeStruct((B, OUT), _f32),
    scratch_shapes=[pltpu.VMEM((4 * B, H), _f32)],
)


@jax.jit
def kernel(x_cell, x_global, pos_cell, edge_index, batch_cell, batch_global,
           Ws0, Wn0, Wgl0, bc0, Gi0, Gs0, bg0,
           Ws1, Wn1, Wgl1, bc1, Gi1, Gs1, bg1,
           l1W, l1b, l2W, l2b, l3W, l3b):
  src = edge_index[0]
  dst = edge_index[1]
  bcell = batch_cell.reshape(N, 1)

  msg0, degp = _msg_deg_kernel(x_cell.reshape(2 * N, HD), src, dst)
  h1, g1 = _dense_call(x_cell, msg0, degp, bcell, x_global,
                       Ws0, Wn0, Wgl0, bc0.reshape(1, H),
                       Gi0, Gs0, bg0.reshape(1, H))
  msg1 = _msg_kernel(h1.reshape(2 * N, HD), src, dst)[0]
  h2, g2 = _dense_call(h1, msg1, degp, bcell, g1,
                       Ws1, Wn1, Wgl1, bc1.reshape(1, H),
                       Gi1, Gs1, bg1.reshape(1, H))
  return _head_call(h2, pos_cell, bcell, g2,
                    l1W, l1b.reshape(1, 2 * H), l2W, l2b.reshape(1, H),
                    l3W, l3b.reshape(1, OUT))


# speedup vs baseline: 11.4120x; 1.1297x over previous
import functools

import jax
import jax.numpy as jnp
from jax import lax
from jax.experimental import pallas as pl
from jax.experimental.pallas import tpu as pltpu
from jax.experimental.pallas import tpu_sc as plsc

N = 10000
E = 320000
B = 32
D = 128
H = 128
OUT = 2

NC = 2
NS = 16
EPT = E // NS
CH = 128
NFULL = EPT // CH
TAIL = EPT - NFULL * CH
RPT = 624
RTAIL = N - RPT * NS
DW = 16
HD = D // 2
NBUF = 6
OUTER = NFULL // NBUF

_f32 = jnp.float32


def _msg_body(with_deg, *refs):
  it = iter(refs)
  table = next(it)
  src = next(it)
  dst = next(it)
  out_msg = next(it)
  out_deg = next(it) if with_deg else None
  idx_s = [next(it) for _ in range(NBUF)]
  idx_d = [next(it) for _ in range(NBUF)]
  idx_g = [next(it) for _ in range(NBUF)]
  rows = [next(it) for _ in range(NBUF)]
  idx_s2 = next(it)
  idx_d2 = next(it)
  idx_g2 = next(it)
  ones = next(it)
  zbuf = next(it)
  zdeg = next(it)
  acc = next(it)
  dacc = next(it) if with_deg else None
  sem_i = [next(it) for _ in range(NBUF)]
  sem_g = [next(it) for _ in range(NBUF)]
  sem_c = [next(it) for _ in range(NBUF)]
  sem_t = next(it)

  cid = lax.axis_index("c")
  sid = lax.axis_index("s")

  z16 = jnp.zeros((16,), _f32)
  o16 = jnp.ones((16,), _f32)

  @pl.loop(0, CH)
  def _(r):
    @pl.loop(0, HD // 16)
    def _(c):
      zbuf[r, pl.ds(c * 16, 16)] = z16

  @pl.loop(0, CH)
  def _(r):
    zdeg[r, :] = z16
    ones[r, :] = o16

  rbase = sid * RPT
  ebase = sid * EPT

  def start_idx(b, c):
    off = pl.multiple_of(ebase + c * CH, 8)
    pltpu.async_copy(src.at[pl.ds(off, CH)], idx_s[b], sem_i[b])
    pltpu.async_copy(dst.at[pl.ds(off, CH)], idx_d[b], sem_i[b])

  def wait_idx(b):
    pltpu.make_async_copy(src.at[pl.ds(0, CH)], idx_s[b], sem_i[b]).wait()
    pltpu.make_async_copy(dst.at[pl.ds(0, CH)], idx_d[b], sem_i[b]).wait()

  for b in range(NBUF):
    start_idx(b, b)

  @pl.loop(0, 4)
  def _(j):
    pltpu.sync_copy(zbuf, acc.at[pl.ds(rbase + j * CH, CH)])

  pltpu.sync_copy(zbuf.at[pl.ds(0, RPT - 4 * CH)],
                  acc.at[pl.ds(rbase + 4 * CH, RPT - 4 * CH)])

  if with_deg:
    @pl.when(cid == 0)
    def _():
      @pl.loop(0, 4)
      def _(j):
        pltpu.sync_copy(zdeg, dacc.at[pl.ds(rbase + j * CH, CH)])

      pltpu.sync_copy(zdeg.at[pl.ds(0, RPT - 4 * CH)],
                      dacc.at[pl.ds(rbase + 4 * CH, RPT - 4 * CH)])

  @pl.when(sid == 0)
  def _():
    pltpu.sync_copy(zbuf.at[pl.ds(0, RTAIL)], acc.at[pl.ds(RPT * NS, RTAIL)])
    if with_deg:
      @pl.when(cid == 0)
      def _():
        pltpu.sync_copy(zdeg.at[pl.ds(0, RTAIL)],
                        dacc.at[pl.ds(RPT * NS, RTAIL)])

  plsc.subcore_barrier()

  @pl.loop(0, OUTER)
  def _(i):
    base_c = i * NBUF
    for b in range(NBUF):
      wait_idx(b)

      @pl.loop(0, CH // 16)
      def _(j):
        idx_g[b][pl.ds(j * 16, 16)] = idx_s[b][pl.ds(j * 16, 16)] * 2 + cid

      pltpu.async_copy(table.at[idx_g[b]], rows[b], sem_g[b])
    for b in range(NBUF):
      pltpu.make_async_copy(table.at[pl.ds(0, CH)], rows[b], sem_g[b]).wait()
      pltpu.async_copy(rows[b], acc.at[idx_d[b]], sem_c[b], add=True)
      if with_deg:
        @pl.when(cid == 0)
        def _():
          pltpu.async_copy(ones, dacc.at[idx_d[b]], sem_c[b], add=True)
    for b in range(NBUF):
      pltpu.make_async_copy(table.at[pl.ds(0, CH)], rows[b], sem_c[b]).wait()
      if with_deg:
        @pl.when(cid == 0)
        def _():
          pltpu.make_async_copy(out_deg.at[pl.ds(0, CH)], ones,
                                sem_c[b]).wait()
      nc = base_c + NBUF + b

      @pl.when(nc < NFULL)
      def _():
        start_idx(b, nc)

  toff = pl.multiple_of(ebase + NFULL * CH, 8)
  pltpu.sync_copy(src.at[pl.ds(toff, TAIL)], idx_s2)
  pltpu.sync_copy(dst.at[pl.ds(toff, TAIL)], idx_d2)

  @pl.loop(0, TAIL // 16)
  def _(j):
    idx_g2[pl.ds(j * 16, 16)] = idx_s2[pl.ds(j * 16, 16)] * 2 + cid

  pltpu.async_copy(table.at[idx_g2], rows[0].at[pl.ds(0, TAIL)],
                   sem_t).wait()
  pltpu.sync_copy(rows[0].at[pl.ds(0, TAIL)], acc.at[idx_d2], add=True)
  if with_deg:
    @pl.when(cid == 0)
    def _():
      pltpu.sync_copy(ones.at[pl.ds(0, TAIL)], dacc.at[idx_d2], add=True)

  plsc.subcore_barrier()

  pltpu.sync_copy(acc.at[pl.ds(rbase, RPT)],
                  out_msg.at[cid, pl.ds(rbase, RPT)])
  if with_deg:
    @pl.when(cid == 0)
    def _():
      pltpu.sync_copy(dacc.at[pl.ds(rbase, RPT)], out_deg.at[pl.ds(rbase, RPT)])

  @pl.when(sid == 0)
  def _():
    pltpu.sync_copy(acc.at[pl.ds(RPT * NS, RTAIL)],
                    out_msg.at[cid, pl.ds(RPT * NS, RTAIL)])
    if with_deg:
      @pl.when(cid == 0)
      def _():
        pltpu.sync_copy(dacc.at[pl.ds(RPT * NS, RTAIL)],
                        out_deg.at[pl.ds(RPT * NS, RTAIL)])


def _make_msg_kernel(with_deg):
  mesh = plsc.VectorSubcoreMesh(core_axis_name="c", subcore_axis_name="s",
                                num_cores=NC, num_subcores=NS)
  out_type = [jax.ShapeDtypeStruct((NC, N, HD), _f32)]
  if with_deg:
    out_type.append(jax.ShapeDtypeStruct((N, DW), _f32))
  scratch = []
  scratch += [pltpu.VMEM((CH,), jnp.int32) for _ in range(NBUF)]
  scratch += [pltpu.VMEM((CH,), jnp.int32) for _ in range(NBUF)]
  scratch += [pltpu.VMEM((CH,), jnp.int32) for _ in range(NBUF)]
  scratch += [pltpu.VMEM((CH, HD), _f32) for _ in range(NBUF)]
  scratch += [
      pltpu.VMEM((TAIL,), jnp.int32),
      pltpu.VMEM((TAIL,), jnp.int32),
      pltpu.VMEM((TAIL,), jnp.int32),
      pltpu.VMEM((CH, DW), _f32),
      pltpu.VMEM((CH, HD), _f32),
      pltpu.VMEM((CH, DW), _f32),
      pltpu.VMEM_SHARED((N, HD), _f32),
  ]
  if with_deg:
    scratch.append(pltpu.VMEM_SHARED((N, DW), _f32))
  scratch += [pltpu.SemaphoreType.DMA for _ in range(3 * NBUF + 1)]
  return pl.kernel(functools.partial(_msg_body, with_deg),
                   out_type=tuple(out_type), mesh=mesh,
                   scratch_types=scratch,
                   compiler_params=pltpu.CompilerParams(
                       use_tc_tiling_on_sc=False))


_msg_deg_kernel = _make_msg_kernel(True)
_msg_kernel = _make_msg_kernel(False)

BN = 1000
NG = N // BN
_HI = lax.Precision.HIGHEST


def _dense_body(h, msgp, degp, bcell, g, ws, wn, wgl, bc, gi, gs, bg,
                h_new, g_new, pooled_acc, cnt_acc):
  i = pl.program_id(0)
  hv = h[...]
  msg = jnp.concatenate([msgp[0], msgp[1]], axis=1)
  deg = jnp.maximum(degp[:, 0:1], 1.0)
  msg = msg / deg
  gv = g[...]
  gw = jnp.dot(gv, wgl[...], preferred_element_type=_f32)
  s = (bcell[...] == lax.broadcasted_iota(jnp.int32, (BN, B), 1)
       ).astype(_f32)
  hn = (jnp.dot(hv, ws[...], preferred_element_type=_f32)
        + jnp.dot(msg, wn[...], preferred_element_type=_f32)
        + jnp.dot(s, gw, preferred_element_type=_f32, precision=_HI)
        + bc[...])
  hn = jnp.maximum(hn, 0.0)
  h_new[...] = hn

  @pl.when(i == 0)
  def _():
    pooled_acc[...] = jnp.zeros((B, H), _f32)
    cnt_acc[...] = jnp.zeros((B, H), _f32)

  dn = (((0,), (0,)), ((), ()))
  pooled_acc[...] += lax.dot_general(s, hn, dn, preferred_element_type=_f32,
                                     precision=_HI)
  cnt_acc[...] += lax.dot_general(s, jnp.ones((BN, H), _f32), dn,
                                  preferred_element_type=_f32, precision=_HI)

  @pl.when(i == NG - 1)
  def _():
    pooled = pooled_acc[...] / jnp.maximum(cnt_acc[...], 1.0)
    gn = (jnp.dot(pooled, gi[...], preferred_element_type=_f32)
          + jnp.dot(gv, gs[...], preferred_element_type=_f32)
          + bg[...])
    g_new[...] = jnp.maximum(gn, 0.0)


_dense_call = pl.pallas_call(
    _dense_body,
    grid=(NG,),
    in_specs=[
        pl.BlockSpec((BN, D), lambda i: (i, 0)),
        pl.BlockSpec((NC, BN, HD), lambda i: (0, i, 0)),
        pl.BlockSpec((BN, DW), lambda i: (i, 0)),
        pl.BlockSpec((BN, 1), lambda i: (i, 0)),
        pl.BlockSpec((B, H), lambda i: (0, 0)),
        pl.BlockSpec((H, H), lambda i: (0, 0)),
        pl.BlockSpec((H, H), lambda i: (0, 0)),
        pl.BlockSpec((H, H), lambda i: (0, 0)),
        pl.BlockSpec((1, H), lambda i: (0, 0)),
        pl.BlockSpec((H, H), lambda i: (0, 0)),
        pl.BlockSpec((H, H), lambda i: (0, 0)),
        pl.BlockSpec((1, H), lambda i: (0, 0)),
    ],
    out_specs=[
        pl.BlockSpec((BN, H), lambda i: (i, 0)),
        pl.BlockSpec((B, H), lambda i: (0, 0)),
    ],
    out_shape=[
        jax.ShapeDtypeStruct((N, H), _f32),
        jax.ShapeDtypeStruct((B, H), _f32),
    ],
    scratch_shapes=[
        pltpu.VMEM((B, H), _f32),
        pltpu.VMEM((B, H), _f32),
    ],
)


def _dense_head_body(h, msgp, degp, bcell, g, ws, wn, wgl, bc, gi, gs, bg,
                     pos, l1w, l1b, l2w, l2b, l3w, l3b,
                     out, pooled_acc, cnt_acc, r_acc):
  i = pl.program_id(0)
  hv = h[...]
  msg = jnp.concatenate([msgp[0], msgp[1]], axis=1)
  deg = jnp.maximum(degp[:, 0:1], 1.0)
  msg = msg / deg
  gv = g[...]
  gw = jnp.dot(gv, wgl[...], preferred_element_type=_f32)
  s = (bcell[...] == lax.broadcasted_iota(jnp.int32, (BN, B), 1)
       ).astype(_f32)
  hn = (jnp.dot(hv, ws[...], preferred_element_type=_f32)
        + jnp.dot(msg, wn[...], preferred_element_type=_f32)
        + jnp.dot(s, gw, preferred_element_type=_f32, precision=_HI)
        + bc[...])
  hn = jnp.maximum(hn, 0.0)

  p = pos[...]
  x0 = p[:, 0:1] - 600.0
  x1 = p[:, 1:2] - 600.0
  lab = ((x1 - x0 > 0.0).astype(jnp.int32)
         + 2 * (x1 + x0 < 0.0).astype(jnp.int32))
  c = lab * B + bcell[...]
  cm = (c == lax.broadcasted_iota(jnp.int32, (BN, 4 * B), 1)).astype(_f32)

  @pl.when(i == 0)
  def _():
    pooled_acc[...] = jnp.zeros((B, H), _f32)
    cnt_acc[...] = jnp.zeros((B, H), _f32)
    r_acc[...] = jnp.zeros((4 * B, H), _f32)

  dn = (((0,), (0,)), ((), ()))
  pooled_acc[...] += lax.dot_general(s, hn, dn, preferred_element_type=_f32,
                                     precision=_HI)
  cnt_acc[...] += lax.dot_general(s, jnp.ones((BN, H), _f32), dn,
                                  preferred_element_type=_f32, precision=_HI)
  r_acc[...] += lax.dot_general(cm, hn, dn, preferred_element_type=_f32,
                                precision=_HI)

  @pl.when(i == NG - 1)
  def _():
    pooled = pooled_acc[...] / jnp.maximum(cnt_acc[...], 1.0)
    gn = (jnp.dot(pooled, gi[...], preferred_element_type=_f32)
          + jnp.dot(gv, gs[...], preferred_element_type=_f32)
          + bg[...])
    gn = jnp.maximum(gn, 0.0)
    r = r_acc[...]
    acc1 = jnp.dot(gn, l1w[pl.ds(4 * H, H), :], preferred_element_type=_f32)
    for q in range(4):
      acc1 += jnp.dot(r[q * B:(q + 1) * B, :], l1w[pl.ds(q * H, H), :],
                      preferred_element_type=_f32)
    y1 = jnp.maximum(acc1 + l1b[...], 0.0)
    y2 = jnp.maximum(jnp.dot(y1, l2w[...], preferred_element_type=_f32)
                     + l2b[...], 0.0)
    out[...] = (jnp.dot(y2, l3w[...], preferred_element_type=_f32)
                + l3b[...])


_dense_head_call = pl.pallas_call(
    _dense_head_body,
    grid=(NG,),
    in_specs=[
        pl.BlockSpec((BN, D), lambda i: (i, 0)),
        pl.BlockSpec((NC, BN, HD), lambda i: (0, i, 0)),
        pl.BlockSpec((BN, DW), lambda i: (i, 0)),
        pl.BlockSpec((BN, 1), lambda i: (i, 0)),
        pl.BlockSpec((B, H), lambda i: (0, 0)),
        pl.BlockSpec((H, H), lambda i: (0, 0)),
        pl.BlockSpec((H, H), lambda i: (0, 0)),
        pl.BlockSpec((H, H), lambda i: (0, 0)),
        pl.BlockSpec((1, H), lambda i: (0, 0)),
        pl.BlockSpec((H, H), lambda i: (0, 0)),
        pl.BlockSpec((H, H), lambda i: (0, 0)),
        pl.BlockSpec((1, H), lambda i: (0, 0)),
        pl.BlockSpec((BN, 2), lambda i: (i, 0)),
        pl.BlockSpec((5 * H, 2 * H), lambda i: (0, 0)),
        pl.BlockSpec((1, 2 * H), lambda i: (0, 0)),
        pl.BlockSpec((2 * H, H), lambda i: (0, 0)),
        pl.BlockSpec((1, H), lambda i: (0, 0)),
        pl.BlockSpec((H, OUT), lambda i: (0, 0)),
        pl.BlockSpec((1, OUT), lambda i: (0, 0)),
    ],
    out_specs=pl.BlockSpec((B, OUT), lambda i: (0, 0)),
    out_shape=jax.ShapeDtypeStruct((B, OUT), _f32),
    scratch_shapes=[
        pltpu.VMEM((B, H), _f32),
        pltpu.VMEM((B, H), _f32),
        pltpu.VMEM((4 * B, H), _f32),
    ],
)


@jax.jit
def kernel(x_cell, x_global, pos_cell, edge_index, batch_cell, batch_global,
           Ws0, Wn0, Wgl0, bc0, Gi0, Gs0, bg0,
           Ws1, Wn1, Wgl1, bc1, Gi1, Gs1, bg1,
           l1W, l1b, l2W, l2b, l3W, l3b):
  src = edge_index[0]
  dst = edge_index[1]
  bcell = batch_cell.reshape(N, 1)

  msg0, deg = _msg_deg_kernel(x_cell.reshape(2 * N, HD), src, dst)
  h1, g1 = _dense_call(x_cell, msg0, deg, bcell, x_global,
                       Ws0, Wn0, Wgl0, bc0.reshape(1, H),
                       Gi0, Gs0, bg0.reshape(1, H))
  msg1 = _msg_kernel(h1.reshape(2 * N, HD), src, dst)[0]
  return _dense_head_call(h1, msg1, deg, bcell, g1,
                          Ws1, Wn1, Wgl1, bc1.reshape(1, H),
                          Gi1, Gs1, bg1.reshape(1, H),
                          pos_cell, l1W, l1b.reshape(1, 2 * H),
                          l2W, l2b.reshape(1, H), l3W, l3b.reshape(1, OUT))


# speedup vs baseline: 11.7936x; 1.0334x over previous
import functools

import jax
import jax.numpy as jnp
from jax import lax
from jax.experimental import pallas as pl
from jax.experimental.pallas import tpu as pltpu
from jax.experimental.pallas import tpu_sc as plsc

N = 10000
E = 320000
B = 32
D = 128
H = 128
OUT = 2

NC = 2
NS = 16
EPT = E // NS
CH = 128
NFULL = EPT // CH
TAIL = EPT - NFULL * CH
RPT = 624
RTAIL = N - RPT * NS
DW = 16
HD = D // 2
NBUF = 8
OUTER = NFULL // NBUF
NREM = NFULL - OUTER * NBUF

_f32 = jnp.float32


def _msg_body(with_deg, *refs):
  it = iter(refs)
  table = next(it)
  src = next(it)
  dst = next(it)
  out_msg = next(it)
  out_deg = next(it) if with_deg else None
  idx_s = [next(it) for _ in range(NBUF)]
  idx_d = [next(it) for _ in range(NBUF)]
  idx_g = [next(it) for _ in range(NBUF)]
  rows = [next(it) for _ in range(NBUF)]
  idx_s2 = next(it)
  idx_d2 = next(it)
  idx_g2 = next(it)
  ones = next(it)
  zbuf = next(it)
  zdeg = next(it)
  acc = next(it)
  dacc = next(it) if with_deg else None
  sem_i = [next(it) for _ in range(NBUF)]
  sem_g = [next(it) for _ in range(NBUF)]
  sem_c = [next(it) for _ in range(NBUF)]
  sem_t = next(it)

  cid = lax.axis_index("c")
  sid = lax.axis_index("s")

  z16 = jnp.zeros((16,), _f32)
  o16 = jnp.ones((16,), _f32)

  @pl.loop(0, CH)
  def _(r):
    @pl.loop(0, HD // 16)
    def _(c):
      zbuf[r, pl.ds(c * 16, 16)] = z16

  @pl.loop(0, CH)
  def _(r):
    zdeg[r, :] = z16
    ones[r, :] = o16

  rbase = sid * RPT
  ebase = sid * EPT

  def start_idx(b, c):
    off = pl.multiple_of(ebase + c * CH, 8)
    pltpu.async_copy(src.at[pl.ds(off, CH)], idx_s[b], sem_i[b])
    pltpu.async_copy(dst.at[pl.ds(off, CH)], idx_d[b], sem_i[b])

  def wait_idx(b):
    pltpu.make_async_copy(src.at[pl.ds(0, CH)], idx_s[b], sem_i[b]).wait()
    pltpu.make_async_copy(dst.at[pl.ds(0, CH)], idx_d[b], sem_i[b]).wait()

  for b in range(NBUF):
    start_idx(b, b)

  @pl.loop(0, 4)
  def _(j):
    pltpu.sync_copy(zbuf, acc.at[pl.ds(rbase + j * CH, CH)])

  pltpu.sync_copy(zbuf.at[pl.ds(0, RPT - 4 * CH)],
                  acc.at[pl.ds(rbase + 4 * CH, RPT - 4 * CH)])

  if with_deg:
    @pl.when(cid == 0)
    def _():
      @pl.loop(0, 4)
      def _(j):
        pltpu.sync_copy(zdeg, dacc.at[pl.ds(rbase + j * CH, CH)])

      pltpu.sync_copy(zdeg.at[pl.ds(0, RPT - 4 * CH)],
                      dacc.at[pl.ds(rbase + 4 * CH, RPT - 4 * CH)])

  @pl.when(sid == 0)
  def _():
    pltpu.sync_copy(zbuf.at[pl.ds(0, RTAIL)], acc.at[pl.ds(RPT * NS, RTAIL)])
    if with_deg:
      @pl.when(cid == 0)
      def _():
        pltpu.sync_copy(zdeg.at[pl.ds(0, RTAIL)],
                        dacc.at[pl.ds(RPT * NS, RTAIL)])

  plsc.subcore_barrier()

  @pl.loop(0, OUTER)
  def _(i):
    base_c = i * NBUF
    for b in range(NBUF):
      wait_idx(b)

      @pl.loop(0, CH // 16)
      def _(j):
        idx_g[b][pl.ds(j * 16, 16)] = idx_s[b][pl.ds(j * 16, 16)] * 2 + cid

      pltpu.async_copy(table.at[idx_g[b]], rows[b], sem_g[b])
    for b in range(NBUF):
      pltpu.make_async_copy(table.at[pl.ds(0, CH)], rows[b], sem_g[b]).wait()
      pltpu.async_copy(rows[b], acc.at[idx_d[b]], sem_c[b], add=True)
      if with_deg:
        @pl.when(cid == 0)
        def _():
          pltpu.async_copy(ones, dacc.at[idx_d[b]], sem_c[b], add=True)
    for b in range(NBUF):
      pltpu.make_async_copy(table.at[pl.ds(0, CH)], rows[b], sem_c[b]).wait()
      if with_deg:
        @pl.when(cid == 0)
        def _():
          pltpu.make_async_copy(out_deg.at[pl.ds(0, CH)], ones,
                                sem_c[b]).wait()
      nc = base_c + NBUF + b

      @pl.when(nc < NFULL)
      def _():
        start_idx(b, nc)

  for b in range(NREM):
    wait_idx(b)

    @pl.loop(0, CH // 16)
    def _(j):
      idx_g[b][pl.ds(j * 16, 16)] = idx_s[b][pl.ds(j * 16, 16)] * 2 + cid

    pltpu.async_copy(table.at[idx_g[b]], rows[b], sem_g[b])
  for b in range(NREM):
    pltpu.make_async_copy(table.at[pl.ds(0, CH)], rows[b], sem_g[b]).wait()
    pltpu.async_copy(rows[b], acc.at[idx_d[b]], sem_c[b], add=True)
    if with_deg:
      @pl.when(cid == 0)
      def _():
        pltpu.async_copy(ones, dacc.at[idx_d[b]], sem_c[b], add=True)
  for b in range(NREM):
    pltpu.make_async_copy(table.at[pl.ds(0, CH)], rows[b], sem_c[b]).wait()
    if with_deg:
      @pl.when(cid == 0)
      def _():
        pltpu.make_async_copy(out_deg.at[pl.ds(0, CH)], ones, sem_c[b]).wait()

  toff = pl.multiple_of(ebase + NFULL * CH, 8)
  pltpu.sync_copy(src.at[pl.ds(toff, TAIL)], idx_s2)
  pltpu.sync_copy(dst.at[pl.ds(toff, TAIL)], idx_d2)

  @pl.loop(0, TAIL // 16)
  def _(j):
    idx_g2[pl.ds(j * 16, 16)] = idx_s2[pl.ds(j * 16, 16)] * 2 + cid

  pltpu.async_copy(table.at[idx_g2], rows[0].at[pl.ds(0, TAIL)],
                   sem_t).wait()
  pltpu.sync_copy(rows[0].at[pl.ds(0, TAIL)], acc.at[idx_d2], add=True)
  if with_deg:
    @pl.when(cid == 0)
    def _():
      pltpu.sync_copy(ones.at[pl.ds(0, TAIL)], dacc.at[idx_d2], add=True)

  plsc.subcore_barrier()

  pltpu.sync_copy(acc.at[pl.ds(rbase, RPT)],
                  out_msg.at[cid, pl.ds(rbase, RPT)])
  if with_deg:
    @pl.when(cid == 0)
    def _():
      pltpu.sync_copy(dacc.at[pl.ds(rbase, RPT)], out_deg.at[pl.ds(rbase, RPT)])

  @pl.when(sid == 0)
  def _():
    pltpu.sync_copy(acc.at[pl.ds(RPT * NS, RTAIL)],
                    out_msg.at[cid, pl.ds(RPT * NS, RTAIL)])
    if with_deg:
      @pl.when(cid == 0)
      def _():
        pltpu.sync_copy(dacc.at[pl.ds(RPT * NS, RTAIL)],
                        out_deg.at[pl.ds(RPT * NS, RTAIL)])


def _make_msg_kernel(with_deg):
  mesh = plsc.VectorSubcoreMesh(core_axis_name="c", subcore_axis_name="s",
                                num_cores=NC, num_subcores=NS)
  out_type = [jax.ShapeDtypeStruct((NC, N, HD), _f32)]
  if with_deg:
    out_type.append(jax.ShapeDtypeStruct((N, DW), _f32))
  scratch = []
  scratch += [pltpu.VMEM((CH,), jnp.int32) for _ in range(NBUF)]
  scratch += [pltpu.VMEM((CH,), jnp.int32) for _ in range(NBUF)]
  scratch += [pltpu.VMEM((CH,), jnp.int32) for _ in range(NBUF)]
  scratch += [pltpu.VMEM((CH, HD), _f32) for _ in range(NBUF)]
  scratch += [
      pltpu.VMEM((TAIL,), jnp.int32),
      pltpu.VMEM((TAIL,), jnp.int32),
      pltpu.VMEM((TAIL,), jnp.int32),
      pltpu.VMEM((CH, DW), _f32),
      pltpu.VMEM((CH, HD), _f32),
      pltpu.VMEM((CH, DW), _f32),
      pltpu.VMEM_SHARED((N, HD), _f32),
  ]
  if with_deg:
    scratch.append(pltpu.VMEM_SHARED((N, DW), _f32))
  scratch += [pltpu.SemaphoreType.DMA for _ in range(3 * NBUF + 1)]
  return pl.kernel(functools.partial(_msg_body, with_deg),
                   out_type=tuple(out_type), mesh=mesh,
                   scratch_types=scratch,
                   compiler_params=pltpu.CompilerParams(
                       use_tc_tiling_on_sc=False))


_msg_deg_kernel = _make_msg_kernel(True)
_msg_kernel = _make_msg_kernel(False)

BN = 1000
NG = N // BN
_HI = lax.Precision.HIGHEST


def _dense_body(h, msgp, degp, bcell, g, ws, wn, wgl, bc, gi, gs, bg,
                h_new, g_new, pooled_acc, cnt_acc):
  i = pl.program_id(0)
  hv = h[...]
  msg = jnp.concatenate([msgp[0], msgp[1]], axis=1)
  deg = jnp.maximum(degp[:, 0:1], 1.0)
  msg = msg / deg
  gv = g[...]
  gw = jnp.dot(gv, wgl[...], preferred_element_type=_f32)
  s = (bcell[...] == lax.broadcasted_iota(jnp.int32, (BN, B), 1)
       ).astype(_f32)
  hn = (jnp.dot(hv, ws[...], preferred_element_type=_f32)
        + jnp.dot(msg, wn[...], preferred_element_type=_f32)
        + jnp.dot(s, gw, preferred_element_type=_f32, precision=_HI)
        + bc[...])
  hn = jnp.maximum(hn, 0.0)
  h_new[...] = hn

  @pl.when(i == 0)
  def _():
    pooled_acc[...] = jnp.zeros((B, H), _f32)
    cnt_acc[...] = jnp.zeros((B, H), _f32)

  dn = (((0,), (0,)), ((), ()))
  pooled_acc[...] += lax.dot_general(s, hn, dn, preferred_element_type=_f32,
                                     precision=_HI)
  cnt_acc[...] += lax.dot_general(s, jnp.ones((BN, H), _f32), dn,
                                  preferred_element_type=_f32, precision=_HI)

  @pl.when(i == NG - 1)
  def _():
    pooled = pooled_acc[...] / jnp.maximum(cnt_acc[...], 1.0)
    gn = (jnp.dot(pooled, gi[...], preferred_element_type=_f32)
          + jnp.dot(gv, gs[...], preferred_element_type=_f32)
          + bg[...])
    g_new[...] = jnp.maximum(gn, 0.0)


_dense_call = pl.pallas_call(
    _dense_body,
    grid=(NG,),
    in_specs=[
        pl.BlockSpec((BN, D), lambda i: (i, 0)),
        pl.BlockSpec((NC, BN, HD), lambda i: (0, i, 0)),
        pl.BlockSpec((BN, DW), lambda i: (i, 0)),
        pl.BlockSpec((BN, 1), lambda i: (i, 0)),
        pl.BlockSpec((B, H), lambda i: (0, 0)),
        pl.BlockSpec((H, H), lambda i: (0, 0)),
        pl.BlockSpec((H, H), lambda i: (0, 0)),
        pl.BlockSpec((H, H), lambda i: (0, 0)),
        pl.BlockSpec((1, H), lambda i: (0, 0)),
        pl.BlockSpec((H, H), lambda i: (0, 0)),
        pl.BlockSpec((H, H), lambda i: (0, 0)),
        pl.BlockSpec((1, H), lambda i: (0, 0)),
    ],
    out_specs=[
        pl.BlockSpec((BN, H), lambda i: (i, 0)),
        pl.BlockSpec((B, H), lambda i: (0, 0)),
    ],
    out_shape=[
        jax.ShapeDtypeStruct((N, H), _f32),
        jax.ShapeDtypeStruct((B, H), _f32),
    ],
    scratch_shapes=[
        pltpu.VMEM((B, H), _f32),
        pltpu.VMEM((B, H), _f32),
    ],
)


def _dense_head_body(h, msgp, degp, bcell, g, ws, wn, wgl, bc, gi, gs, bg,
                     pos, l1w, l1b, l2w, l2b, l3w, l3b,
                     out, pooled_acc, cnt_acc, r_acc):
  i = pl.program_id(0)
  hv = h[...]
  msg = jnp.concatenate([msgp[0], msgp[1]], axis=1)
  deg = jnp.maximum(degp[:, 0:1], 1.0)
  msg = msg / deg
  gv = g[...]
  gw = jnp.dot(gv, wgl[...], preferred_element_type=_f32)
  s = (bcell[...] == lax.broadcasted_iota(jnp.int32, (BN, B), 1)
       ).astype(_f32)
  hn = (jnp.dot(hv, ws[...], preferred_element_type=_f32)
        + jnp.dot(msg, wn[...], preferred_element_type=_f32)
        + jnp.dot(s, gw, preferred_element_type=_f32, precision=_HI)
        + bc[...])
  hn = jnp.maximum(hn, 0.0)

  p = pos[...]
  x0 = p[:, 0:1] - 600.0
  x1 = p[:, 1:2] - 600.0
  lab = ((x1 - x0 > 0.0).astype(jnp.int32)
         + 2 * (x1 + x0 < 0.0).astype(jnp.int32))
  c = lab * B + bcell[...]
  cm = (c == lax.broadcasted_iota(jnp.int32, (BN, 4 * B), 1)).astype(_f32)

  @pl.when(i == 0)
  def _():
    pooled_acc[...] = jnp.zeros((B, H), _f32)
    cnt_acc[...] = jnp.zeros((B, H), _f32)
    r_acc[...] = jnp.zeros((4 * B, H), _f32)

  dn = (((0,), (0,)), ((), ()))
  pooled_acc[...] += lax.dot_general(s, hn, dn, preferred_element_type=_f32,
                                     precision=_HI)
  cnt_acc[...] += lax.dot_general(s, jnp.ones((BN, H), _f32), dn,
                                  preferred_element_type=_f32, precision=_HI)
  r_acc[...] += lax.dot_general(cm, hn, dn, preferred_element_type=_f32,
                                precision=_HI)

  @pl.when(i == NG - 1)
  def _():
    pooled = pooled_acc[...] / jnp.maximum(cnt_acc[...], 1.0)
    gn = (jnp.dot(pooled, gi[...], preferred_element_type=_f32)
          + jnp.dot(gv, gs[...], preferred_element_type=_f32)
          + bg[...])
    gn = jnp.maximum(gn, 0.0)
    r = r_acc[...]
    acc1 = jnp.dot(gn, l1w[pl.ds(4 * H, H), :], preferred_element_type=_f32)
    for q in range(4):
      acc1 += jnp.dot(r[q * B:(q + 1) * B, :], l1w[pl.ds(q * H, H), :],
                      preferred_element_type=_f32)
    y1 = jnp.maximum(acc1 + l1b[...], 0.0)
    y2 = jnp.maximum(jnp.dot(y1, l2w[...], preferred_element_type=_f32)
                     + l2b[...], 0.0)
    out[...] = (jnp.dot(y2, l3w[...], preferred_element_type=_f32)
                + l3b[...])


_dense_head_call = pl.pallas_call(
    _dense_head_body,
    grid=(NG,),
    in_specs=[
        pl.BlockSpec((BN, D), lambda i: (i, 0)),
        pl.BlockSpec((NC, BN, HD), lambda i: (0, i, 0)),
        pl.BlockSpec((BN, DW), lambda i: (i, 0)),
        pl.BlockSpec((BN, 1), lambda i: (i, 0)),
        pl.BlockSpec((B, H), lambda i: (0, 0)),
        pl.BlockSpec((H, H), lambda i: (0, 0)),
        pl.BlockSpec((H, H), lambda i: (0, 0)),
        pl.BlockSpec((H, H), lambda i: (0, 0)),
        pl.BlockSpec((1, H), lambda i: (0, 0)),
        pl.BlockSpec((H, H), lambda i: (0, 0)),
        pl.BlockSpec((H, H), lambda i: (0, 0)),
        pl.BlockSpec((1, H), lambda i: (0, 0)),
        pl.BlockSpec((BN, 2), lambda i: (i, 0)),
        pl.BlockSpec((5 * H, 2 * H), lambda i: (0, 0)),
        pl.BlockSpec((1, 2 * H), lambda i: (0, 0)),
        pl.BlockSpec((2 * H, H), lambda i: (0, 0)),
        pl.BlockSpec((1, H), lambda i: (0, 0)),
        pl.BlockSpec((H, OUT), lambda i: (0, 0)),
        pl.BlockSpec((1, OUT), lambda i: (0, 0)),
    ],
    out_specs=pl.BlockSpec((B, OUT), lambda i: (0, 0)),
    out_shape=jax.ShapeDtypeStruct((B, OUT), _f32),
    scratch_shapes=[
        pltpu.VMEM((B, H), _f32),
        pltpu.VMEM((B, H), _f32),
        pltpu.VMEM((4 * B, H), _f32),
    ],
)


@jax.jit
def kernel(x_cell, x_global, pos_cell, edge_index, batch_cell, batch_global,
           Ws0, Wn0, Wgl0, bc0, Gi0, Gs0, bg0,
           Ws1, Wn1, Wgl1, bc1, Gi1, Gs1, bg1,
           l1W, l1b, l2W, l2b, l3W, l3b):
  src = edge_index[0]
  dst = edge_index[1]
  bcell = batch_cell.reshape(N, 1)

  msg0, deg = _msg_deg_kernel(x_cell.reshape(2 * N, HD), src, dst)
  h1, g1 = _dense_call(x_cell, msg0, deg, bcell, x_global,
                       Ws0, Wn0, Wgl0, bc0.reshape(1, H),
                       Gi0, Gs0, bg0.reshape(1, H))
  msg1 = _msg_kernel(h1.reshape(2 * N, HD), src, dst)[0]
  return _dense_head_call(h1, msg1, deg, bcell, g1,
                          Ws1, Wn1, Wgl1, bc1.reshape(1, H),
                          Gi1, Gs1, bg1.reshape(1, H),
                          pos_cell, l1W, l1b.reshape(1, 2 * H),
                          l2W, l2b.reshape(1, H), l3W, l3b.reshape(1, OUT))


# speedup vs baseline: 11.9562x; 1.0138x over previous
import functools

import jax
import jax.numpy as jnp
from jax import lax
from jax.experimental import pallas as pl
from jax.experimental.pallas import tpu as pltpu
from jax.experimental.pallas import tpu_sc as plsc

N = 10000
E = 320000
B = 32
D = 128
H = 128
OUT = 2

NC = 2
NS = 16
EPT = E // NS
CH = 128
NFULL = EPT // CH
TAIL = EPT - NFULL * CH
RPT = 624
RTAIL = N - RPT * NS
DW = 16
HD = D // 2
NBUF = 8
OUTER = NFULL // NBUF
NREM = NFULL - OUTER * NBUF

_f32 = jnp.float32


def _msg_body(with_deg, *refs):
  it = iter(refs)
  table = next(it)
  src = next(it)
  dst = next(it)
  out_msg = next(it)
  out_deg = next(it) if with_deg else None
  idx_s = [next(it) for _ in range(NBUF)]
  idx_d = [next(it) for _ in range(NBUF)]
  idx_g = [next(it) for _ in range(NBUF)]
  rows = [next(it) for _ in range(NBUF)]
  idx_s2 = next(it)
  idx_d2 = next(it)
  idx_g2 = next(it)
  ones = next(it)
  zbuf = next(it)
  zdeg = next(it)
  acc = next(it)
  dacc = next(it) if with_deg else None
  sem_i = [next(it) for _ in range(NBUF)]
  sem_g = [next(it) for _ in range(NBUF)]
  sem_c = [next(it) for _ in range(NBUF)]
  sem_t = next(it)

  cid = lax.axis_index("c")
  sid = lax.axis_index("s")

  z16 = jnp.zeros((16,), _f32)
  o16 = jnp.ones((16,), _f32)

  @pl.loop(0, CH)
  def _(r):
    @pl.loop(0, HD // 16)
    def _(c):
      zbuf[r, pl.ds(c * 16, 16)] = z16

  @pl.loop(0, CH)
  def _(r):
    zdeg[r, :] = z16
    ones[r, :] = o16

  rbase = sid * RPT
  ebase = sid * EPT

  def start_idx(b, c):
    off = pl.multiple_of(ebase + c * CH, 8)
    pltpu.async_copy(src.at[pl.ds(off, CH)], idx_s[b], sem_i[b])
    pltpu.async_copy(dst.at[pl.ds(off, CH)], idx_d[b], sem_i[b])

  def wait_idx(b):
    pltpu.make_async_copy(src.at[pl.ds(0, CH)], idx_s[b], sem_i[b]).wait()
    pltpu.make_async_copy(dst.at[pl.ds(0, CH)], idx_d[b], sem_i[b]).wait()

  for b in range(NBUF):
    start_idx(b, b)

  @pl.loop(0, 4)
  def _(j):
    pltpu.sync_copy(zbuf, acc.at[pl.ds(rbase + j * CH, CH)])

  pltpu.sync_copy(zbuf.at[pl.ds(0, RPT - 4 * CH)],
                  acc.at[pl.ds(rbase + 4 * CH, RPT - 4 * CH)])

  if with_deg:
    @pl.loop(0, 4)
    def _(j):
      pltpu.sync_copy(zdeg, dacc.at[pl.ds(rbase + j * CH, CH)])

    pltpu.sync_copy(zdeg.at[pl.ds(0, RPT - 4 * CH)],
                    dacc.at[pl.ds(rbase + 4 * CH, RPT - 4 * CH)])

  @pl.when(sid == 0)
  def _():
    pltpu.sync_copy(zbuf.at[pl.ds(0, RTAIL)], acc.at[pl.ds(RPT * NS, RTAIL)])
    if with_deg:
      pltpu.sync_copy(zdeg.at[pl.ds(0, RTAIL)],
                      dacc.at[pl.ds(RPT * NS, RTAIL)])

  plsc.subcore_barrier()

  @pl.loop(0, OUTER)
  def _(i):
    base_c = i * NBUF
    for b in range(NBUF):
      wait_idx(b)

      @pl.loop(0, CH // 16)
      def _(j):
        idx_g[b][pl.ds(j * 16, 16)] = idx_s[b][pl.ds(j * 16, 16)] * 2 + cid

      pltpu.async_copy(table.at[idx_g[b]], rows[b], sem_g[b])
    for b in range(NBUF):
      pltpu.make_async_copy(table.at[pl.ds(0, CH)], rows[b], sem_g[b]).wait()
      pltpu.async_copy(rows[b], acc.at[idx_d[b]], sem_c[b], add=True)
      if with_deg:
        @pl.when(cid == b % 2)
        def _():
          pltpu.async_copy(ones, dacc.at[idx_d[b]], sem_c[b], add=True)
    for b in range(NBUF):
      pltpu.make_async_copy(table.at[pl.ds(0, CH)], rows[b], sem_c[b]).wait()
      if with_deg:
        @pl.when(cid == b % 2)
        def _():
          pltpu.make_async_copy(out_deg.at[0, pl.ds(0, CH)], ones,
                                sem_c[b]).wait()
      nc = base_c + NBUF + b

      @pl.when(nc < NFULL)
      def _():
        start_idx(b, nc)

  for b in range(NREM):
    wait_idx(b)

    @pl.loop(0, CH // 16)
    def _(j):
      idx_g[b][pl.ds(j * 16, 16)] = idx_s[b][pl.ds(j * 16, 16)] * 2 + cid

    pltpu.async_copy(table.at[idx_g[b]], rows[b], sem_g[b])
  for b in range(NREM):
    pltpu.make_async_copy(table.at[pl.ds(0, CH)], rows[b], sem_g[b]).wait()
    pltpu.async_copy(rows[b], acc.at[idx_d[b]], sem_c[b], add=True)
    if with_deg:
      @pl.when(cid == b % 2)
      def _():
        pltpu.async_copy(ones, dacc.at[idx_d[b]], sem_c[b], add=True)
  for b in range(NREM):
    pltpu.make_async_copy(table.at[pl.ds(0, CH)], rows[b], sem_c[b]).wait()
    if with_deg:
      @pl.when(cid == b % 2)
      def _():
        pltpu.make_async_copy(out_deg.at[0, pl.ds(0, CH)], ones,
                              sem_c[b]).wait()

  toff = pl.multiple_of(ebase + NFULL * CH, 8)
  pltpu.sync_copy(src.at[pl.ds(toff, TAIL)], idx_s2)
  pltpu.sync_copy(dst.at[pl.ds(toff, TAIL)], idx_d2)

  @pl.loop(0, TAIL // 16)
  def _(j):
    idx_g2[pl.ds(j * 16, 16)] = idx_s2[pl.ds(j * 16, 16)] * 2 + cid

  pltpu.async_copy(table.at[idx_g2], rows[0].at[pl.ds(0, TAIL)],
                   sem_t).wait()
  pltpu.sync_copy(rows[0].at[pl.ds(0, TAIL)], acc.at[idx_d2], add=True)
  if with_deg:
    @pl.when(cid == 0)
    def _():
      pltpu.sync_copy(ones.at[pl.ds(0, TAIL)], dacc.at[idx_d2], add=True)

  plsc.subcore_barrier()

  pltpu.sync_copy(acc.at[pl.ds(rbase, RPT)],
                  out_msg.at[cid, pl.ds(rbase, RPT)])
  if with_deg:
    pltpu.sync_copy(dacc.at[pl.ds(rbase, RPT)],
                    out_deg.at[cid, pl.ds(rbase, RPT)])

  @pl.when(sid == 0)
  def _():
    pltpu.sync_copy(acc.at[pl.ds(RPT * NS, RTAIL)],
                    out_msg.at[cid, pl.ds(RPT * NS, RTAIL)])
    if with_deg:
      pltpu.sync_copy(dacc.at[pl.ds(RPT * NS, RTAIL)],
                      out_deg.at[cid, pl.ds(RPT * NS, RTAIL)])


def _make_msg_kernel(with_deg):
  mesh = plsc.VectorSubcoreMesh(core_axis_name="c", subcore_axis_name="s",
                                num_cores=NC, num_subcores=NS)
  out_type = [jax.ShapeDtypeStruct((NC, N, HD), _f32)]
  if with_deg:
    out_type.append(jax.ShapeDtypeStruct((NC, N, DW), _f32))
  scratch = []
  scratch += [pltpu.VMEM((CH,), jnp.int32) for _ in range(NBUF)]
  scratch += [pltpu.VMEM((CH,), jnp.int32) for _ in range(NBUF)]
  scratch += [pltpu.VMEM((CH,), jnp.int32) for _ in range(NBUF)]
  scratch += [pltpu.VMEM((CH, HD), _f32) for _ in range(NBUF)]
  scratch += [
      pltpu.VMEM((TAIL,), jnp.int32),
      pltpu.VMEM((TAIL,), jnp.int32),
      pltpu.VMEM((TAIL,), jnp.int32),
      pltpu.VMEM((CH, DW), _f32),
      pltpu.VMEM((CH, HD), _f32),
      pltpu.VMEM((CH, DW), _f32),
      pltpu.VMEM_SHARED((N, HD), _f32),
  ]
  if with_deg:
    scratch.append(pltpu.VMEM_SHARED((N, DW), _f32))
  scratch += [pltpu.SemaphoreType.DMA for _ in range(3 * NBUF + 1)]
  return pl.kernel(functools.partial(_msg_body, with_deg),
                   out_type=tuple(out_type), mesh=mesh,
                   scratch_types=scratch,
                   compiler_params=pltpu.CompilerParams(
                       use_tc_tiling_on_sc=False))


_msg_deg_kernel = _make_msg_kernel(True)
_msg_kernel = _make_msg_kernel(False)

BN = 1000
NG = N // BN
_HI = lax.Precision.HIGHEST


def _dense_body(h, msgp, degp, bcell, g, ws, wn, wgl, bc, gi, gs, bg,
                h_new, g_new, pooled_acc, cnt_acc):
  i = pl.program_id(0)
  hv = h[...]
  msg = jnp.concatenate([msgp[0], msgp[1]], axis=1)
  deg = jnp.maximum(degp[0, :, 0:1] + degp[1, :, 0:1], 1.0)
  msg = msg / deg
  gv = g[...]
  gw = jnp.dot(gv, wgl[...], preferred_element_type=_f32)
  s = (bcell[...] == lax.broadcasted_iota(jnp.int32, (BN, B), 1)
       ).astype(_f32)
  hn = (jnp.dot(hv, ws[...], preferred_element_type=_f32)
        + jnp.dot(msg, wn[...], preferred_element_type=_f32)
        + jnp.dot(s, gw, preferred_element_type=_f32, precision=_HI)
        + bc[...])
  hn = jnp.maximum(hn, 0.0)
  h_new[...] = hn

  @pl.when(i == 0)
  def _():
    pooled_acc[...] = jnp.zeros((B, H), _f32)
    cnt_acc[...] = jnp.zeros((B, H), _f32)

  dn = (((0,), (0,)), ((), ()))
  pooled_acc[...] += lax.dot_general(s, hn, dn, preferred_element_type=_f32,
                                     precision=_HI)
  cnt_acc[...] += lax.dot_general(s, jnp.ones((BN, H), _f32), dn,
                                  preferred_element_type=_f32)

  @pl.when(i == NG - 1)
  def _():
    pooled = pooled_acc[...] / jnp.maximum(cnt_acc[...], 1.0)
    gn = (jnp.dot(pooled, gi[...], preferred_element_type=_f32)
          + jnp.dot(gv, gs[...], preferred_element_type=_f32)
          + bg[...])
    g_new[...] = jnp.maximum(gn, 0.0)


_dense_call = pl.pallas_call(
    _dense_body,
    grid=(NG,),
    in_specs=[
        pl.BlockSpec((BN, D), lambda i: (i, 0)),
        pl.BlockSpec((NC, BN, HD), lambda i: (0, i, 0)),
        pl.BlockSpec((NC, BN, DW), lambda i: (0, i, 0)),
        pl.BlockSpec((BN, 1), lambda i: (i, 0)),
        pl.BlockSpec((B, H), lambda i: (0, 0)),
        pl.BlockSpec((H, H), lambda i: (0, 0)),
        pl.BlockSpec((H, H), lambda i: (0, 0)),
        pl.BlockSpec((H, H), lambda i: (0, 0)),
        pl.BlockSpec((1, H), lambda i: (0, 0)),
        pl.BlockSpec((H, H), lambda i: (0, 0)),
        pl.BlockSpec((H, H), lambda i: (0, 0)),
        pl.BlockSpec((1, H), lambda i: (0, 0)),
    ],
    out_specs=[
        pl.BlockSpec((BN, H), lambda i: (i, 0)),
        pl.BlockSpec((B, H), lambda i: (0, 0)),
    ],
    out_shape=[
        jax.ShapeDtypeStruct((N, H), _f32),
        jax.ShapeDtypeStruct((B, H), _f32),
    ],
    scratch_shapes=[
        pltpu.VMEM((B, H), _f32),
        pltpu.VMEM((B, H), _f32),
    ],
)


def _dense_head_body(h, msgp, degp, bcell, g, ws, wn, wgl, bc, gi, gs, bg,
                     pos, l1w, l1b, l2w, l2b, l3w, l3b,
                     out, pooled_acc, cnt_acc, r_acc):
  i = pl.program_id(0)
  hv = h[...]
  msg = jnp.concatenate([msgp[0], msgp[1]], axis=1)
  deg = jnp.maximum(degp[0, :, 0:1] + degp[1, :, 0:1], 1.0)
  msg = msg / deg
  gv = g[...]
  gw = jnp.dot(gv, wgl[...], preferred_element_type=_f32)
  s = (bcell[...] == lax.broadcasted_iota(jnp.int32, (BN, B), 1)
       ).astype(_f32)
  hn = (jnp.dot(hv, ws[...], preferred_element_type=_f32)
        + jnp.dot(msg, wn[...], preferred_element_type=_f32)
        + jnp.dot(s, gw, preferred_element_type=_f32, precision=_HI)
        + bc[...])
  hn = jnp.maximum(hn, 0.0)

  p = pos[...]
  x0 = p[:, 0:1] - 600.0
  x1 = p[:, 1:2] - 600.0
  lab = ((x1 - x0 > 0.0).astype(jnp.int32)
         + 2 * (x1 + x0 < 0.0).astype(jnp.int32))
  c = lab * B + bcell[...]
  cm = (c == lax.broadcasted_iota(jnp.int32, (BN, 4 * B), 1)).astype(_f32)

  @pl.when(i == 0)
  def _():
    pooled_acc[...] = jnp.zeros((B, H), _f32)
    cnt_acc[...] = jnp.zeros((B, H), _f32)
    r_acc[...] = jnp.zeros((4 * B, H), _f32)

  dn = (((0,), (0,)), ((), ()))
  pooled_acc[...] += lax.dot_general(s, hn, dn, preferred_element_type=_f32,
                                     precision=_HI)
  cnt_acc[...] += lax.dot_general(s, jnp.ones((BN, H), _f32), dn,
                                  preferred_element_type=_f32)
  r_acc[...] += lax.dot_general(cm, hn, dn, preferred_element_type=_f32,
                                precision=_HI)

  @pl.when(i == NG - 1)
  def _():
    pooled = pooled_acc[...] / jnp.maximum(cnt_acc[...], 1.0)
    gn = (jnp.dot(pooled, gi[...], preferred_element_type=_f32)
          + jnp.dot(gv, gs[...], preferred_element_type=_f32)
          + bg[...])
    gn = jnp.maximum(gn, 0.0)
    r = r_acc[...]
    acc1 = jnp.dot(gn, l1w[pl.ds(4 * H, H), :], preferred_element_type=_f32)
    for q in range(4):
      acc1 += jnp.dot(r[q * B:(q + 1) * B, :], l1w[pl.ds(q * H, H), :],
                      preferred_element_type=_f32)
    y1 = jnp.maximum(acc1 + l1b[...], 0.0)
    y2 = jnp.maximum(jnp.dot(y1, l2w[...], preferred_element_type=_f32)
                     + l2b[...], 0.0)
    out[...] = (jnp.dot(y2, l3w[...], preferred_element_type=_f32)
                + l3b[...])


_dense_head_call = pl.pallas_call(
    _dense_head_body,
    grid=(NG,),
    in_specs=[
        pl.BlockSpec((BN, D), lambda i: (i, 0)),
        pl.BlockSpec((NC, BN, HD), lambda i: (0, i, 0)),
        pl.BlockSpec((NC, BN, DW), lambda i: (0, i, 0)),
        pl.BlockSpec((BN, 1), lambda i: (i, 0)),
        pl.BlockSpec((B, H), lambda i: (0, 0)),
        pl.BlockSpec((H, H), lambda i: (0, 0)),
        pl.BlockSpec((H, H), lambda i: (0, 0)),
        pl.BlockSpec((H, H), lambda i: (0, 0)),
        pl.BlockSpec((1, H), lambda i: (0, 0)),
        pl.BlockSpec((H, H), lambda i: (0, 0)),
        pl.BlockSpec((H, H), lambda i: (0, 0)),
        pl.BlockSpec((1, H), lambda i: (0, 0)),
        pl.BlockSpec((BN, 2), lambda i: (i, 0)),
        pl.BlockSpec((5 * H, 2 * H), lambda i: (0, 0)),
        pl.BlockSpec((1, 2 * H), lambda i: (0, 0)),
        pl.BlockSpec((2 * H, H), lambda i: (0, 0)),
        pl.BlockSpec((1, H), lambda i: (0, 0)),
        pl.BlockSpec((H, OUT), lambda i: (0, 0)),
        pl.BlockSpec((1, OUT), lambda i: (0, 0)),
    ],
    out_specs=pl.BlockSpec((B, OUT), lambda i: (0, 0)),
    out_shape=jax.ShapeDtypeStruct((B, OUT), _f32),
    scratch_shapes=[
        pltpu.VMEM((B, H), _f32),
        pltpu.VMEM((B, H), _f32),
        pltpu.VMEM((4 * B, H), _f32),
    ],
)


@jax.jit
def kernel(x_cell, x_global, pos_cell, edge_index, batch_cell, batch_global,
           Ws0, Wn0, Wgl0, bc0, Gi0, Gs0, bg0,
           Ws1, Wn1, Wgl1, bc1, Gi1, Gs1, bg1,
           l1W, l1b, l2W, l2b, l3W, l3b):
  src = edge_index[0]
  dst = edge_index[1]
  bcell = batch_cell.reshape(N, 1)

  msg0, deg = _msg_deg_kernel(x_cell.reshape(2 * N, HD), src, dst)
  h1, g1 = _dense_call(x_cell, msg0, deg, bcell, x_global,
                       Ws0, Wn0, Wgl0, bc0.reshape(1, H),
                       Gi0, Gs0, bg0.reshape(1, H))
  msg1 = _msg_kernel(h1.reshape(2 * N, HD), src, dst)[0]
  return _dense_head_call(h1, msg1, deg, bcell, g1,
                          Ws1, Wn1, Wgl1, bc1.reshape(1, H),
                          Gi1, Gs1, bg1.reshape(1, H),
                          pos_cell, l1W, l1b.reshape(1, 2 * H),
                          l2W, l2b.reshape(1, H), l3W, l3b.reshape(1, OUT))


# speedup vs baseline: 12.1755x; 1.0183x over previous
import functools

import jax
import jax.numpy as jnp
from jax import lax
from jax.experimental import pallas as pl
from jax.experimental.pallas import tpu as pltpu
from jax.experimental.pallas import tpu_sc as plsc

N = 10000
E = 320000
B = 32
D = 128
H = 128
OUT = 2

NC = 2
NS = 16
EPT = E // NS
CH = 128
NFULL = EPT // CH
TAIL = EPT - NFULL * CH
RPT = 624
RTAIL = N - RPT * NS
DW = 16
HD = D // 2
NBUF = 8
OUTER = NFULL // NBUF
NREM = NFULL - OUTER * NBUF

_f32 = jnp.float32


def _msg_body(with_deg, *refs):
  it = iter(refs)
  table = next(it)
  src = next(it)
  dst = next(it)
  out_msg = next(it)
  out_deg = next(it) if with_deg else None
  idx_s = [next(it) for _ in range(NBUF)]
  idx_d = [next(it) for _ in range(NBUF)]
  idx_g = [next(it) for _ in range(NBUF)]
  rows = [next(it) for _ in range(NBUF)]
  idx_s2 = next(it)
  idx_d2 = next(it)
  idx_g2 = next(it)
  ones = next(it)
  zbuf = next(it)
  zdeg = next(it)
  acc = next(it)
  dacc = next(it) if with_deg else None
  sem_i = [next(it) for _ in range(NBUF)]
  sem_g = [next(it) for _ in range(NBUF)]
  sem_c = [next(it) for _ in range(NBUF)]
  sem_t = next(it)

  cid = lax.axis_index("c")
  sid = lax.axis_index("s")

  z16 = jnp.zeros((16,), _f32)
  o16 = jnp.ones((16,), _f32)

  @pl.loop(0, CH)
  def _(r):
    @pl.loop(0, HD // 16)
    def _(c):
      zbuf[r, pl.ds(c * 16, 16)] = z16

  @pl.loop(0, CH)
  def _(r):
    zdeg[r, :] = z16
    ones[r, :] = o16

  rbase = sid * RPT
  ebase = sid * EPT

  def start_idx(b, c):
    off = pl.multiple_of(ebase + c * CH, 8)
    pltpu.async_copy(src.at[pl.ds(off, CH)], idx_s[b], sem_i[b])
    pltpu.async_copy(dst.at[pl.ds(off, CH)], idx_d[b], sem_i[b])

  def wait_idx(b):
    pltpu.make_async_copy(src.at[pl.ds(0, CH)], idx_s[b], sem_i[b]).wait()
    pltpu.make_async_copy(dst.at[pl.ds(0, CH)], idx_d[b], sem_i[b]).wait()

  for b in range(NBUF):
    start_idx(b, b)

  @pl.loop(0, 4)
  def _(j):
    pltpu.sync_copy(zbuf, acc.at[pl.ds(rbase + j * CH, CH)])

  pltpu.sync_copy(zbuf.at[pl.ds(0, RPT - 4 * CH)],
                  acc.at[pl.ds(rbase + 4 * CH, RPT - 4 * CH)])

  if with_deg:
    @pl.loop(0, 4)
    def _(j):
      pltpu.sync_copy(zdeg, dacc.at[pl.ds(rbase + j * CH, CH)])

    pltpu.sync_copy(zdeg.at[pl.ds(0, RPT - 4 * CH)],
                    dacc.at[pl.ds(rbase + 4 * CH, RPT - 4 * CH)])

  @pl.when(sid == 0)
  def _():
    pltpu.sync_copy(zbuf.at[pl.ds(0, RTAIL)], acc.at[pl.ds(RPT * NS, RTAIL)])
    if with_deg:
      pltpu.sync_copy(zdeg.at[pl.ds(0, RTAIL)],
                      dacc.at[pl.ds(RPT * NS, RTAIL)])

  plsc.subcore_barrier()

  @pl.loop(0, OUTER)
  def _(i):
    base_c = i * NBUF
    for b in range(NBUF):
      wait_idx(b)

      @pl.loop(0, CH // 16)
      def _(j):
        idx_g[b][pl.ds(j * 16, 16)] = idx_s[b][pl.ds(j * 16, 16)] * 2 + cid

      pltpu.async_copy(table.at[idx_g[b]], rows[b], sem_g[b])
    for b in range(NBUF):
      pltpu.make_async_copy(table.at[pl.ds(0, CH)], rows[b], sem_g[b]).wait()
      pltpu.async_copy(rows[b], acc.at[idx_d[b]], sem_c[b], add=True)
      if with_deg:
        @pl.when(cid == b % 2)
        def _():
          pltpu.async_copy(ones, dacc.at[idx_d[b]], sem_c[b], add=True)
    for b in range(NBUF):
      pltpu.make_async_copy(table.at[pl.ds(0, CH)], rows[b], sem_c[b]).wait()
      if with_deg:
        @pl.when(cid == b % 2)
        def _():
          pltpu.make_async_copy(out_deg.at[0, pl.ds(0, CH)], ones,
                                sem_c[b]).wait()
      nc = base_c + NBUF + b

      @pl.when(nc < NFULL)
      def _():
        start_idx(b, nc)

  for b in range(NREM):
    wait_idx(b)

    @pl.loop(0, CH // 16)
    def _(j):
      idx_g[b][pl.ds(j * 16, 16)] = idx_s[b][pl.ds(j * 16, 16)] * 2 + cid

    pltpu.async_copy(table.at[idx_g[b]], rows[b], sem_g[b])
  for b in range(NREM):
    pltpu.make_async_copy(table.at[pl.ds(0, CH)], rows[b], sem_g[b]).wait()
    pltpu.async_copy(rows[b], acc.at[idx_d[b]], sem_c[b], add=True)
    if with_deg:
      @pl.when(cid == b % 2)
      def _():
        pltpu.async_copy(ones, dacc.at[idx_d[b]], sem_c[b], add=True)
  for b in range(NREM):
    pltpu.make_async_copy(table.at[pl.ds(0, CH)], rows[b], sem_c[b]).wait()
    if with_deg:
      @pl.when(cid == b % 2)
      def _():
        pltpu.make_async_copy(out_deg.at[0, pl.ds(0, CH)], ones,
                              sem_c[b]).wait()

  toff = pl.multiple_of(ebase + NFULL * CH, 8)
  pltpu.sync_copy(src.at[pl.ds(toff, TAIL)], idx_s2)
  pltpu.sync_copy(dst.at[pl.ds(toff, TAIL)], idx_d2)

  @pl.loop(0, TAIL // 16)
  def _(j):
    idx_g2[pl.ds(j * 16, 16)] = idx_s2[pl.ds(j * 16, 16)] * 2 + cid

  pltpu.async_copy(table.at[idx_g2], rows[0].at[pl.ds(0, TAIL)],
                   sem_t).wait()
  pltpu.sync_copy(rows[0].at[pl.ds(0, TAIL)], acc.at[idx_d2], add=True)
  if with_deg:
    @pl.when(cid == 0)
    def _():
      pltpu.sync_copy(ones.at[pl.ds(0, TAIL)], dacc.at[idx_d2], add=True)

  plsc.subcore_barrier()

  pltpu.sync_copy(acc.at[pl.ds(rbase, RPT)],
                  out_msg.at[cid, pl.ds(rbase, RPT)])
  if with_deg:
    pltpu.sync_copy(dacc.at[pl.ds(rbase, RPT)],
                    out_deg.at[cid, pl.ds(rbase, RPT)])

  @pl.when(sid == 0)
  def _():
    pltpu.sync_copy(acc.at[pl.ds(RPT * NS, RTAIL)],
                    out_msg.at[cid, pl.ds(RPT * NS, RTAIL)])
    if with_deg:
      pltpu.sync_copy(dacc.at[pl.ds(RPT * NS, RTAIL)],
                      out_deg.at[cid, pl.ds(RPT * NS, RTAIL)])


def _make_msg_kernel(with_deg):
  mesh = plsc.VectorSubcoreMesh(core_axis_name="c", subcore_axis_name="s",
                                num_cores=NC, num_subcores=NS)
  out_type = [jax.ShapeDtypeStruct((NC, N, HD), _f32)]
  if with_deg:
    out_type.append(jax.ShapeDtypeStruct((NC, N, DW), _f32))
  scratch = []
  scratch += [pltpu.VMEM((CH,), jnp.int32) for _ in range(NBUF)]
  scratch += [pltpu.VMEM((CH,), jnp.int32) for _ in range(NBUF)]
  scratch += [pltpu.VMEM((CH,), jnp.int32) for _ in range(NBUF)]
  scratch += [pltpu.VMEM((CH, HD), _f32) for _ in range(NBUF)]
  scratch += [
      pltpu.VMEM((TAIL,), jnp.int32),
      pltpu.VMEM((TAIL,), jnp.int32),
      pltpu.VMEM((TAIL,), jnp.int32),
      pltpu.VMEM((CH, DW), _f32),
      pltpu.VMEM((CH, HD), _f32),
      pltpu.VMEM((CH, DW), _f32),
      pltpu.VMEM_SHARED((N, HD), _f32),
  ]
  if with_deg:
    scratch.append(pltpu.VMEM_SHARED((N, DW), _f32))
  scratch += [pltpu.SemaphoreType.DMA for _ in range(3 * NBUF + 1)]
  return pl.kernel(functools.partial(_msg_body, with_deg),
                   out_type=tuple(out_type), mesh=mesh,
                   scratch_types=scratch,
                   compiler_params=pltpu.CompilerParams(
                       use_tc_tiling_on_sc=False))


_msg_deg_kernel = _make_msg_kernel(True)
_msg_kernel = _make_msg_kernel(False)

BN = 1000
NG = N // BN
_HI = lax.Precision.HIGHEST


def _pre_body(h, bcell, g, ws, wgl, bc, t_out):
  gw = jnp.dot(g[...], wgl[...], preferred_element_type=_f32)
  s = (bcell[...] == lax.broadcasted_iota(jnp.int32, (BN, B), 1)
       ).astype(_f32)
  t_out[...] = (jnp.dot(h[...], ws[...], preferred_element_type=_f32)
                + jnp.dot(s, gw, preferred_element_type=_f32, precision=_HI)
                + bc[...])


_pre_call = pl.pallas_call(
    _pre_body,
    grid=(NG,),
    in_specs=[
        pl.BlockSpec((BN, D), lambda i: (i, 0)),
        pl.BlockSpec((BN, 1), lambda i: (i, 0)),
        pl.BlockSpec((B, H), lambda i: (0, 0)),
        pl.BlockSpec((H, H), lambda i: (0, 0)),
        pl.BlockSpec((H, H), lambda i: (0, 0)),
        pl.BlockSpec((1, H), lambda i: (0, 0)),
    ],
    out_specs=pl.BlockSpec((BN, H), lambda i: (i, 0)),
    out_shape=jax.ShapeDtypeStruct((N, H), _f32),
)


def _finish_hn(t, msgp, degp, wn):
  msg = jnp.concatenate([msgp[0], msgp[1]], axis=1)
  deg = jnp.maximum(degp[0, :, 0:1] + degp[1, :, 0:1], 1.0)
  return jnp.maximum(
      t[...] + jnp.dot(msg / deg, wn[...], preferred_element_type=_f32), 0.0)


def _mid_body(t, msgp, degp, bcell, g, wn, gi, gs, bg,
              h_new, g_new, pooled_acc, cnt_acc):
  i = pl.program_id(0)
  hn = _finish_hn(t, msgp, degp, wn)
  h_new[...] = hn
  s = (bcell[...] == lax.broadcasted_iota(jnp.int32, (BN, B), 1)
       ).astype(_f32)

  @pl.when(i == 0)
  def _():
    pooled_acc[...] = jnp.zeros((B, H), _f32)
    cnt_acc[...] = jnp.zeros((B, H), _f32)

  dn = (((0,), (0,)), ((), ()))
  pooled_acc[...] += lax.dot_general(s, hn, dn, preferred_element_type=_f32,
                                     precision=_HI)
  cnt_acc[...] += lax.dot_general(s, jnp.ones((BN, H), _f32), dn,
                                  preferred_element_type=_f32)

  @pl.when(i == NG - 1)
  def _():
    pooled = pooled_acc[...] / jnp.maximum(cnt_acc[...], 1.0)
    gn = (jnp.dot(pooled, gi[...], preferred_element_type=_f32)
          + jnp.dot(g[...], gs[...], preferred_element_type=_f32)
          + bg[...])
    g_new[...] = jnp.maximum(gn, 0.0)


_mid_call = pl.pallas_call(
    _mid_body,
    grid=(NG,),
    in_specs=[
        pl.BlockSpec((BN, H), lambda i: (i, 0)),
        pl.BlockSpec((NC, BN, HD), lambda i: (0, i, 0)),
        pl.BlockSpec((NC, BN, DW), lambda i: (0, i, 0)),
        pl.BlockSpec((BN, 1), lambda i: (i, 0)),
        pl.BlockSpec((B, H), lambda i: (0, 0)),
        pl.BlockSpec((H, H), lambda i: (0, 0)),
        pl.BlockSpec((H, H), lambda i: (0, 0)),
        pl.BlockSpec((H, H), lambda i: (0, 0)),
        pl.BlockSpec((1, H), lambda i: (0, 0)),
    ],
    out_specs=[
        pl.BlockSpec((BN, H), lambda i: (i, 0)),
        pl.BlockSpec((B, H), lambda i: (0, 0)),
    ],
    out_shape=[
        jax.ShapeDtypeStruct((N, H), _f32),
        jax.ShapeDtypeStruct((B, H), _f32),
    ],
    scratch_shapes=[
        pltpu.VMEM((B, H), _f32),
        pltpu.VMEM((B, H), _f32),
    ],
)


def _mid_head_body(t, msgp, degp, bcell, g, wn, gi, gs, bg,
                   pos, l1w, l1b, l2w, l2b, l3w, l3b,
                   out, pooled_acc, cnt_acc, r_acc):
  i = pl.program_id(0)
  hn = _finish_hn(t, msgp, degp, wn)
  s = (bcell[...] == lax.broadcasted_iota(jnp.int32, (BN, B), 1)
       ).astype(_f32)

  p = pos[...]
  x0 = p[:, 0:1] - 600.0
  x1 = p[:, 1:2] - 600.0
  lab = ((x1 - x0 > 0.0).astype(jnp.int32)
         + 2 * (x1 + x0 < 0.0).astype(jnp.int32))
  c = lab * B + bcell[...]
  cm = (c == lax.broadcasted_iota(jnp.int32, (BN, 4 * B), 1)).astype(_f32)

  @pl.when(i == 0)
  def _():
    pooled_acc[...] = jnp.zeros((B, H), _f32)
    cnt_acc[...] = jnp.zeros((B, H), _f32)
    r_acc[...] = jnp.zeros((4 * B, H), _f32)

  dn = (((0,), (0,)), ((), ()))
  pooled_acc[...] += lax.dot_general(s, hn, dn, preferred_element_type=_f32,
                                     precision=_HI)
  cnt_acc[...] += lax.dot_general(s, jnp.ones((BN, H), _f32), dn,
                                  preferred_element_type=_f32)
  r_acc[...] += lax.dot_general(cm, hn, dn, preferred_element_type=_f32,
                                precision=_HI)

  @pl.when(i == NG - 1)
  def _():
    pooled = pooled_acc[...] / jnp.maximum(cnt_acc[...], 1.0)
    gn = (jnp.dot(pooled, gi[...], preferred_element_type=_f32)
          + jnp.dot(g[...], gs[...], preferred_element_type=_f32)
          + bg[...])
    gn = jnp.maximum(gn, 0.0)
    r = r_acc[...]
    acc1 = jnp.dot(gn, l1w[pl.ds(4 * H, H), :], preferred_element_type=_f32)
    for q in range(4):
      acc1 += jnp.dot(r[q * B:(q + 1) * B, :], l1w[pl.ds(q * H, H), :],
                      preferred_element_type=_f32)
    y1 = jnp.maximum(acc1 + l1b[...], 0.0)
    y2 = jnp.maximum(jnp.dot(y1, l2w[...], preferred_element_type=_f32)
                     + l2b[...], 0.0)
    out[...] = (jnp.dot(y2, l3w[...], preferred_element_type=_f32)
                + l3b[...])


_mid_head_call = pl.pallas_call(
    _mid_head_body,
    grid=(NG,),
    in_specs=[
        pl.BlockSpec((BN, H), lambda i: (i, 0)),
        pl.BlockSpec((NC, BN, HD), lambda i: (0, i, 0)),
        pl.BlockSpec((NC, BN, DW), lambda i: (0, i, 0)),
        pl.BlockSpec((BN, 1), lambda i: (i, 0)),
        pl.BlockSpec((B, H), lambda i: (0, 0)),
        pl.BlockSpec((H, H), lambda i: (0, 0)),
        pl.BlockSpec((H, H), lambda i: (0, 0)),
        pl.BlockSpec((H, H), lambda i: (0, 0)),
        pl.BlockSpec((1, H), lambda i: (0, 0)),
        pl.BlockSpec((BN, 2), lambda i: (i, 0)),
        pl.BlockSpec((5 * H, 2 * H), lambda i: (0, 0)),
        pl.BlockSpec((1, 2 * H), lambda i: (0, 0)),
        pl.BlockSpec((2 * H, H), lambda i: (0, 0)),
        pl.BlockSpec((1, H), lambda i: (0, 0)),
        pl.BlockSpec((H, OUT), lambda i: (0, 0)),
        pl.BlockSpec((1, OUT), lambda i: (0, 0)),
    ],
    out_specs=pl.BlockSpec((B, OUT), lambda i: (0, 0)),
    out_shape=jax.ShapeDtypeStruct((B, OUT), _f32),
    scratch_shapes=[
        pltpu.VMEM((B, H), _f32),
        pltpu.VMEM((B, H), _f32),
        pltpu.VMEM((4 * B, H), _f32),
    ],
)


@jax.jit
def kernel(x_cell, x_global, pos_cell, edge_index, batch_cell, batch_global,
           Ws0, Wn0, Wgl0, bc0, Gi0, Gs0, bg0,
           Ws1, Wn1, Wgl1, bc1, Gi1, Gs1, bg1,
           l1W, l1b, l2W, l2b, l3W, l3b):
  src = edge_index[0]
  dst = edge_index[1]
  bcell = batch_cell.reshape(N, 1)

  msg0, deg = _msg_deg_kernel(x_cell.reshape(2 * N, HD), src, dst)
  t0 = _pre_call(x_cell, bcell, x_global, Ws0, Wgl0, bc0.reshape(1, H))
  h1, g1 = _mid_call(t0, msg0, deg, bcell, x_global,
                     Wn0, Gi0, Gs0, bg0.reshape(1, H))
  msg1 = _msg_kernel(h1.reshape(2 * N, HD), src, dst)[0]
  t1 = _pre_call(h1, bcell, g1, Ws1, Wgl1, bc1.reshape(1, H))
  return _mid_head_call(t1, msg1, deg, bcell, g1,
                        Wn1, Gi1, Gs1, bg1.reshape(1, H),
                        pos_cell, l1W, l1b.reshape(1, 2 * H),
                        l2W, l2b.reshape(1, H), l3W, l3b.reshape(1, OUT))


# speedup vs baseline: 12.5407x; 1.0300x over previous
import functools

import jax
import jax.numpy as jnp
from jax import lax
from jax.experimental import pallas as pl
from jax.experimental.pallas import tpu as pltpu
from jax.experimental.pallas import tpu_sc as plsc

N = 10000
E = 320000
B = 32
D = 128
H = 128
OUT = 2

NC = 2
NS = 16
EPT = E // NS
CH = 128
NFULL = EPT // CH
TAIL = EPT - NFULL * CH
RPT = 624
RTAIL = N - RPT * NS
DW = 16
HD = D // 2
NBUF = 8
OUTER = NFULL // NBUF
NREM = NFULL - OUTER * NBUF

_f32 = jnp.float32


def _msg_body(with_deg, *refs):
  it = iter(refs)
  table = next(it)
  eidx = next(it)
  out_msg = next(it)
  out_deg = next(it) if with_deg else None
  idx_sd = [next(it) for _ in range(NBUF)]
  idx_g = [next(it) for _ in range(NBUF)]
  rows = [next(it) for _ in range(NBUF)]
  idx_sd2 = next(it)
  idx_g2 = next(it)
  ones = next(it)
  zbuf = next(it)
  zdeg = next(it)
  acc = next(it)
  dacc = next(it) if with_deg else None
  sem_i = [next(it) for _ in range(NBUF)]
  sem_g = [next(it) for _ in range(NBUF)]
  sem_c = [next(it) for _ in range(NBUF)]
  sem_t = next(it)

  cid = lax.axis_index("c")
  sid = lax.axis_index("s")

  z16 = jnp.zeros((16,), _f32)
  o16 = jnp.ones((16,), _f32)

  @pl.loop(0, CH)
  def _(r):
    @pl.loop(0, HD // 16)
    def _(c):
      zbuf[r, pl.ds(c * 16, 16)] = z16

  @pl.loop(0, CH)
  def _(r):
    zdeg[r, :] = z16
    ones[r, :] = o16

  rbase = sid * RPT
  ebase = sid * EPT

  def start_idx(b, c):
    off = pl.multiple_of(ebase + c * CH, 8)
    pltpu.async_copy(eidx.at[:, pl.ds(off, CH)], idx_sd[b], sem_i[b])

  def wait_idx(b):
    pltpu.make_async_copy(eidx.at[:, pl.ds(0, CH)], idx_sd[b],
                          sem_i[b]).wait()

  for b in range(NBUF):
    start_idx(b, b)

  @pl.loop(0, 4)
  def _(j):
    pltpu.sync_copy(zbuf, acc.at[pl.ds(rbase + j * CH, CH)])

  pltpu.sync_copy(zbuf.at[pl.ds(0, RPT - 4 * CH)],
                  acc.at[pl.ds(rbase + 4 * CH, RPT - 4 * CH)])

  if with_deg:
    @pl.loop(0, 4)
    def _(j):
      pltpu.sync_copy(zdeg, dacc.at[pl.ds(rbase + j * CH, CH)])

    pltpu.sync_copy(zdeg.at[pl.ds(0, RPT - 4 * CH)],
                    dacc.at[pl.ds(rbase + 4 * CH, RPT - 4 * CH)])

  @pl.when(sid == 0)
  def _():
    pltpu.sync_copy(zbuf.at[pl.ds(0, RTAIL)], acc.at[pl.ds(RPT * NS, RTAIL)])
    if with_deg:
      pltpu.sync_copy(zdeg.at[pl.ds(0, RTAIL)],
                      dacc.at[pl.ds(RPT * NS, RTAIL)])

  plsc.subcore_barrier()

  @pl.loop(0, OUTER)
  def _(i):
    base_c = i * NBUF
    for b in range(NBUF):
      wait_idx(b)

      @pl.loop(0, CH // 16)
      def _(j):
        idx_g[b][pl.ds(j * 16, 16)] = idx_sd[b][0, pl.ds(j * 16, 16)] * 2 + cid

      pltpu.async_copy(table.at[idx_g[b]], rows[b], sem_g[b])
    for b in range(NBUF):
      pltpu.make_async_copy(table.at[pl.ds(0, CH)], rows[b], sem_g[b]).wait()
      pltpu.async_copy(rows[b], acc.at[idx_sd[b].at[1]], sem_c[b], add=True)
      if with_deg:
        @pl.when(cid == b % 2)
        def _():
          pltpu.async_copy(ones, dacc.at[idx_sd[b].at[1]], sem_c[b], add=True)
    for b in range(NBUF):
      pltpu.make_async_copy(table.at[pl.ds(0, CH)], rows[b], sem_c[b]).wait()
      if with_deg:
        @pl.when(cid == b % 2)
        def _():
          pltpu.make_async_copy(out_deg.at[0, pl.ds(0, CH)], ones,
                                sem_c[b]).wait()
      nc = base_c + NBUF + b

      @pl.when(nc < NFULL)
      def _():
        start_idx(b, nc)

  for b in range(NREM):
    wait_idx(b)

    @pl.loop(0, CH // 16)
    def _(j):
      idx_g[b][pl.ds(j * 16, 16)] = idx_sd[b][0, pl.ds(j * 16, 16)] * 2 + cid

    pltpu.async_copy(table.at[idx_g[b]], rows[b], sem_g[b])
  for b in range(NREM):
    pltpu.make_async_copy(table.at[pl.ds(0, CH)], rows[b], sem_g[b]).wait()
    pltpu.async_copy(rows[b], acc.at[idx_sd[b].at[1]], sem_c[b], add=True)
    if with_deg:
      @pl.when(cid == b % 2)
      def _():
        pltpu.async_copy(ones, dacc.at[idx_sd[b].at[1]], sem_c[b], add=True)
  for b in range(NREM):
    pltpu.make_async_copy(table.at[pl.ds(0, CH)], rows[b], sem_c[b]).wait()
    if with_deg:
      @pl.when(cid == b % 2)
      def _():
        pltpu.make_async_copy(out_deg.at[0, pl.ds(0, CH)], ones,
                              sem_c[b]).wait()

  toff = pl.multiple_of(ebase + NFULL * CH, 8)
  pltpu.sync_copy(eidx.at[:, pl.ds(toff, TAIL)], idx_sd2)

  @pl.loop(0, TAIL // 16)
  def _(j):
    idx_g2[pl.ds(j * 16, 16)] = idx_sd2[0, pl.ds(j * 16, 16)] * 2 + cid

  pltpu.async_copy(table.at[idx_g2], rows[0].at[pl.ds(0, TAIL)],
                   sem_t).wait()
  pltpu.sync_copy(rows[0].at[pl.ds(0, TAIL)], acc.at[idx_sd2.at[1]], add=True)
  if with_deg:
    @pl.when(cid == 0)
    def _():
      pltpu.sync_copy(ones.at[pl.ds(0, TAIL)], dacc.at[idx_sd2.at[1]], add=True)

  plsc.subcore_barrier()

  pltpu.sync_copy(acc.at[pl.ds(rbase, RPT)],
                  out_msg.at[cid, pl.ds(rbase, RPT)])
  if with_deg:
    pltpu.sync_copy(dacc.at[pl.ds(rbase, RPT)],
                    out_deg.at[cid, pl.ds(rbase, RPT)])

  @pl.when(sid == 0)
  def _():
    pltpu.sync_copy(acc.at[pl.ds(RPT * NS, RTAIL)],
                    out_msg.at[cid, pl.ds(RPT * NS, RTAIL)])
    if with_deg:
      pltpu.sync_copy(dacc.at[pl.ds(RPT * NS, RTAIL)],
                      out_deg.at[cid, pl.ds(RPT * NS, RTAIL)])


def _make_msg_kernel(with_deg):
  mesh = plsc.VectorSubcoreMesh(core_axis_name="c", subcore_axis_name="s",
                                num_cores=NC, num_subcores=NS)
  out_type = [jax.ShapeDtypeStruct((NC, N, HD), _f32)]
  if with_deg:
    out_type.append(jax.ShapeDtypeStruct((NC, N, DW), _f32))
  scratch = []
  scratch += [pltpu.VMEM((2, CH), jnp.int32) for _ in range(NBUF)]
  scratch += [pltpu.VMEM((CH,), jnp.int32) for _ in range(NBUF)]
  scratch += [pltpu.VMEM((CH, HD), _f32) for _ in range(NBUF)]
  scratch += [
      pltpu.VMEM((2, TAIL), jnp.int32),
      pltpu.VMEM((TAIL,), jnp.int32),
      pltpu.VMEM((CH, DW), _f32),
      pltpu.VMEM((CH, HD), _f32),
      pltpu.VMEM((CH, DW), _f32),
      pltpu.VMEM_SHARED((N, HD), _f32),
  ]
  if with_deg:
    scratch.append(pltpu.VMEM_SHARED((N, DW), _f32))
  scratch += [pltpu.SemaphoreType.DMA for _ in range(3 * NBUF + 1)]
  return pl.kernel(functools.partial(_msg_body, with_deg),
                   out_type=tuple(out_type), mesh=mesh,
                   scratch_types=scratch,
                   compiler_params=pltpu.CompilerParams(
                       use_tc_tiling_on_sc=False))


_msg_deg_kernel = _make_msg_kernel(True)
_msg_kernel = _make_msg_kernel(False)

BN = 1000
NG = N // BN
_HI = lax.Precision.HIGHEST


def _pre_body(h, bcell, g, ws, wgl, bc, t_out):
  gw = jnp.dot(g[...], wgl[...], preferred_element_type=_f32)
  s = (bcell[...] == lax.broadcasted_iota(jnp.int32, (BN, B), 1)
       ).astype(_f32)
  t_out[...] = (jnp.dot(h[...], ws[...], preferred_element_type=_f32)
                + jnp.dot(s, gw, preferred_element_type=_f32, precision=_HI)
                + bc[...])


_pre_call = pl.pallas_call(
    _pre_body,
    grid=(NG,),
    in_specs=[
        pl.BlockSpec((BN, D), lambda i: (i, 0)),
        pl.BlockSpec((BN, 1), lambda i: (i, 0)),
        pl.BlockSpec((B, H), lambda i: (0, 0)),
        pl.BlockSpec((H, H), lambda i: (0, 0)),
        pl.BlockSpec((H, H), lambda i: (0, 0)),
        pl.BlockSpec((1, H), lambda i: (0, 0)),
    ],
    out_specs=pl.BlockSpec((BN, H), lambda i: (i, 0)),
    out_shape=jax.ShapeDtypeStruct((N, H), _f32),
)


def _finish_hn(t, msgp, degp, wn):
  msg = jnp.concatenate([msgp[0], msgp[1]], axis=1)
  deg = jnp.maximum(degp[0, :, 0:1] + degp[1, :, 0:1], 1.0)
  return jnp.maximum(
      t[...] + jnp.dot(msg / deg, wn[...], preferred_element_type=_f32), 0.0)


def _mid_body(t, msgp, degp, bcell, g, wn, gi, gs, bg,
              h_new, g_new, pooled_acc, cnt_acc):
  i = pl.program_id(0)
  hn = _finish_hn(t, msgp, degp, wn)
  h_new[...] = hn
  s = (bcell[...] == lax.broadcasted_iota(jnp.int32, (BN, B), 1)
       ).astype(_f32)

  @pl.when(i == 0)
  def _():
    pooled_acc[...] = jnp.zeros((B, H), _f32)
    cnt_acc[...] = jnp.zeros((B, H), _f32)

  dn = (((0,), (0,)), ((), ()))
  pooled_acc[...] += lax.dot_general(s, hn, dn, preferred_element_type=_f32,
                                     precision=_HI)
  cnt_acc[...] += lax.dot_general(s, jnp.ones((BN, H), _f32), dn,
                                  preferred_element_type=_f32)

  @pl.when(i == NG - 1)
  def _():
    pooled = pooled_acc[...] / jnp.maximum(cnt_acc[...], 1.0)
    gn = (jnp.dot(pooled, gi[...], preferred_element_type=_f32)
          + jnp.dot(g[...], gs[...], preferred_element_type=_f32)
          + bg[...])
    g_new[...] = jnp.maximum(gn, 0.0)


_mid_call = pl.pallas_call(
    _mid_body,
    grid=(NG,),
    in_specs=[
        pl.BlockSpec((BN, H), lambda i: (i, 0)),
        pl.BlockSpec((NC, BN, HD), lambda i: (0, i, 0)),
        pl.BlockSpec((NC, BN, DW), lambda i: (0, i, 0)),
        pl.BlockSpec((BN, 1), lambda i: (i, 0)),
        pl.BlockSpec((B, H), lambda i: (0, 0)),
        pl.BlockSpec((H, H), lambda i: (0, 0)),
        pl.BlockSpec((H, H), lambda i: (0, 0)),
        pl.BlockSpec((H, H), lambda i: (0, 0)),
        pl.BlockSpec((1, H), lambda i: (0, 0)),
    ],
    out_specs=[
        pl.BlockSpec((BN, H), lambda i: (i, 0)),
        pl.BlockSpec((B, H), lambda i: (0, 0)),
    ],
    out_shape=[
        jax.ShapeDtypeStruct((N, H), _f32),
        jax.ShapeDtypeStruct((B, H), _f32),
    ],
    scratch_shapes=[
        pltpu.VMEM((B, H), _f32),
        pltpu.VMEM((B, H), _f32),
    ],
)


def _mid_head_body(t, msgp, degp, bcell, g, wn, gi, gs, bg,
                   pos, l1w, l1b, l2w, l2b, l3w, l3b,
                   out, pooled_acc, cnt_acc, r_acc):
  i = pl.program_id(0)
  hn = _finish_hn(t, msgp, degp, wn)
  s = (bcell[...] == lax.broadcasted_iota(jnp.int32, (BN, B), 1)
       ).astype(_f32)

  p = pos[...]
  x0 = p[:, 0:1] - 600.0
  x1 = p[:, 1:2] - 600.0
  lab = ((x1 - x0 > 0.0).astype(jnp.int32)
         + 2 * (x1 + x0 < 0.0).astype(jnp.int32))
  c = lab * B + bcell[...]
  cm = (c == lax.broadcasted_iota(jnp.int32, (BN, 4 * B), 1)).astype(_f32)

  @pl.when(i == 0)
  def _():
    pooled_acc[...] = jnp.zeros((B, H), _f32)
    cnt_acc[...] = jnp.zeros((B, H), _f32)
    r_acc[...] = jnp.zeros((4 * B, H), _f32)

  dn = (((0,), (0,)), ((), ()))
  pooled_acc[...] += lax.dot_general(s, hn, dn, preferred_element_type=_f32,
                                     precision=_HI)
  cnt_acc[...] += lax.dot_general(s, jnp.ones((BN, H), _f32), dn,
                                  preferred_element_type=_f32)
  r_acc[...] += lax.dot_general(cm, hn, dn, preferred_element_type=_f32,
                                precision=_HI)

  @pl.when(i == NG - 1)
  def _():
    pooled = pooled_acc[...] / jnp.maximum(cnt_acc[...], 1.0)
    gn = (jnp.dot(pooled, gi[...], preferred_element_type=_f32)
          + jnp.dot(g[...], gs[...], preferred_element_type=_f32)
          + bg[...])
    gn = jnp.maximum(gn, 0.0)
    r = r_acc[...]
    acc1 = jnp.dot(gn, l1w[pl.ds(4 * H, H), :], preferred_element_type=_f32)
    for q in range(4):
      acc1 += jnp.dot(r[q * B:(q + 1) * B, :], l1w[pl.ds(q * H, H), :],
                      preferred_element_type=_f32)
    y1 = jnp.maximum(acc1 + l1b[...], 0.0)
    y2 = jnp.maximum(jnp.dot(y1, l2w[...], preferred_element_type=_f32)
                     + l2b[...], 0.0)
    out[...] = (jnp.dot(y2, l3w[...], preferred_element_type=_f32)
                + l3b[...])


_mid_head_call = pl.pallas_call(
    _mid_head_body,
    grid=(NG,),
    in_specs=[
        pl.BlockSpec((BN, H), lambda i: (i, 0)),
        pl.BlockSpec((NC, BN, HD), lambda i: (0, i, 0)),
        pl.BlockSpec((NC, BN, DW), lambda i: (0, i, 0)),
        pl.BlockSpec((BN, 1), lambda i: (i, 0)),
        pl.BlockSpec((B, H), lambda i: (0, 0)),
        pl.BlockSpec((H, H), lambda i: (0, 0)),
        pl.BlockSpec((H, H), lambda i: (0, 0)),
        pl.BlockSpec((H, H), lambda i: (0, 0)),
        pl.BlockSpec((1, H), lambda i: (0, 0)),
        pl.BlockSpec((BN, 2), lambda i: (i, 0)),
        pl.BlockSpec((5 * H, 2 * H), lambda i: (0, 0)),
        pl.BlockSpec((1, 2 * H), lambda i: (0, 0)),
        pl.BlockSpec((2 * H, H), lambda i: (0, 0)),
        pl.BlockSpec((1, H), lambda i: (0, 0)),
        pl.BlockSpec((H, OUT), lambda i: (0, 0)),
        pl.BlockSpec((1, OUT), lambda i: (0, 0)),
    ],
    out_specs=pl.BlockSpec((B, OUT), lambda i: (0, 0)),
    out_shape=jax.ShapeDtypeStruct((B, OUT), _f32),
    scratch_shapes=[
        pltpu.VMEM((B, H), _f32),
        pltpu.VMEM((B, H), _f32),
        pltpu.VMEM((4 * B, H), _f32),
    ],
)


@jax.jit
def kernel(x_cell, x_global, pos_cell, edge_index, batch_cell, batch_global,
           Ws0, Wn0, Wgl0, bc0, Gi0, Gs0, bg0,
           Ws1, Wn1, Wgl1, bc1, Gi1, Gs1, bg1,
           l1W, l1b, l2W, l2b, l3W, l3b):
  bcell = batch_cell.reshape(N, 1)

  msg0, deg = _msg_deg_kernel(x_cell.reshape(2 * N, HD), edge_index)
  t0 = _pre_call(x_cell, bcell, x_global, Ws0, Wgl0, bc0.reshape(1, H))
  h1, g1 = _mid_call(t0, msg0, deg, bcell, x_global,
                     Wn0, Gi0, Gs0, bg0.reshape(1, H))
  msg1 = _msg_kernel(h1.reshape(2 * N, HD), edge_index)[0]
  t1 = _pre_call(h1, bcell, g1, Ws1, Wgl1, bc1.reshape(1, H))
  return _mid_head_call(t1, msg1, deg, bcell, g1,
                        Wn1, Gi1, Gs1, bg1.reshape(1, H),
                        pos_cell, l1W, l1b.reshape(1, 2 * H),
                        l2W, l2b.reshape(1, H), l3W, l3b.reshape(1, OUT))
